# baseline (speedup 1.0000x reference)
import functools

import jax
import jax.numpy as jnp
from jax import lax
from jax.experimental import pallas as pl
from jax.experimental.pallas import tpu as pltpu
from jax.experimental.pallas import tpu_sc as plsc

_NUM_CLASSES = 21
_N = 20000
_CHUNK = 640
_NV = _CHUNK // 16
_G = 64
_MAX_POS = 32
_N_BG = 96
_TOTAL = 128
_RECI = 64
_RECF = 528
_NW = 32
_OUTW = _NUM_CLASSES * 4
_LASTW = 4 * (_N - _CHUNK * (_NW - 1))

_mesh = plsc.VectorSubcoreMesh(core_axis_name="c", subcore_axis_name="s",
                               num_cores=2)
_cparams = pltpu.CompilerParams(needs_layout_passes=False)


def _vlog(x):
    bits = plsc.bitcast(x, jnp.int32)
    e = ((bits >> 23) & 0xFF) - 127
    m = plsc.bitcast((bits & 0x7FFFFF) | 0x3F800000, jnp.float32)
    big = m > 1.4142135623730951
    m = jnp.where(big, m * 0.5, m)
    e = jnp.where(big, e + 1, e)
    r = (m - 1.0) / (m + 1.0)
    r2 = r * r
    p = 1.0 / 7.0 + r2 * (1.0 / 9.0)
    p = 1.0 / 5.0 + r2 * p
    p = 1.0 / 3.0 + r2 * p
    p = 1.0 + r2 * p
    return e.astype(jnp.float32) * 0.6931471805599453 + 2.0 * r * p


@functools.partial(
    pl.kernel,
    out_type=[
        jax.ShapeDtypeStruct((_NW * _RECI,), jnp.int32),
        jax.ShapeDtypeStruct((_NW * _RECF,), jnp.float32),
    ],
    mesh=_mesh,
    compiler_params=_cparams,
    scratch_types=[
        pltpu.VMEM((4 * _CHUNK,), jnp.float32),
        pltpu.VMEM((4 * _G,), jnp.float32),
        pltpu.VMEM((_G,), jnp.float32),
        pltpu.VMEM((_CHUNK,), jnp.float32),
        pltpu.VMEM((_CHUNK,), jnp.float32),
        pltpu.VMEM((_CHUNK,), jnp.int32),
        pltpu.VMEM((_RECI,), jnp.int32),
        pltpu.VMEM((_RECF,), jnp.float32),
    ],
)
def _phase1(rois_hbm, gt_hbm, reci_hbm, recf_hbm,
            rv_v, gt_v, a2_v, bi_v, bu_v, ag_v, ri_v, rf_v):
    wid = lax.axis_index("s") * 2 + lax.axis_index("c")
    base = wid * _CHUNK

    @pl.when(wid < _NW - 1)
    def _():
        pltpu.sync_copy(rois_hbm.at[pl.ds(base * 4, 4 * _CHUNK)], rv_v)

    @pl.when(wid == _NW - 1)
    def _():
        pltpu.sync_copy(rois_hbm.at[pl.ds(base * 4, _LASTW)],
                        rv_v.at[pl.ds(0, _LASTW)])

    pltpu.sync_copy(gt_hbm, gt_v)

    iota = lax.iota(jnp.int32, 16)
    for k in range(_G // 16):
        i4 = (iota + k * 16) * 4
        gx1 = plsc.load_gather(gt_v, [i4])
        gy1 = plsc.load_gather(gt_v, [i4 + 1])
        gx2 = plsc.load_gather(gt_v, [i4 + 2])
        gy2 = plsc.load_gather(gt_v, [i4 + 3])
        a2_v[pl.ds(k * 16, 16)] = (gx2 - gx1) * (gy2 - gy1)

    _T = 4

    def vec_body(q, _):
        off = q * (16 * _T)
        x1s, y1s, x2s, y2s, a1s = [], [], [], [], []
        for t in range(_T):
            i4 = (iota + off + t * 16) * 4
            x1 = plsc.load_gather(rv_v, [i4])
            y1 = plsc.load_gather(rv_v, [i4 + 1])
            x2 = plsc.load_gather(rv_v, [i4 + 2])
            y2 = plsc.load_gather(rv_v, [i4 + 3])
            x1s.append(x1)
            y1s.append(y1)
            x2s.append(x2)
            y2s.append(y2)
            a1s.append((x2 - x1) * (y2 - y1))

        def gt_body(j, carry):
            bis, bus, bas = carry
            js = jnp.full((16,), j, dtype=jnp.int32)
            j4 = jnp.full((16,), j * 4, dtype=jnp.int32)
            gx1 = plsc.load_gather(gt_v, [j4])
            gy1 = plsc.load_gather(gt_v, [j4 + 1])
            gx2 = plsc.load_gather(gt_v, [j4 + 2])
            gy2 = plsc.load_gather(gt_v, [j4 + 3])
            a2 = plsc.load_gather(a2_v, [js])
            nbi, nbu, nba = [], [], []
            for t in range(_T):
                w = jnp.minimum(x2s[t], gx2) - jnp.maximum(x1s[t], gx1)
                h = jnp.minimum(y2s[t], gy2) - jnp.maximum(y1s[t], gy1)
                inter = jnp.maximum(w, 0.0) * jnp.maximum(h, 0.0)
                union = a1s[t] + a2 - inter
                better = inter * bus[t] > bis[t] * union
                nbi.append(jnp.where(better, inter, bis[t]))
                nbu.append(jnp.where(better, union, bus[t]))
                nba.append(jnp.where(better, js, bas[t]))
            return tuple(nbi), tuple(nbu), tuple(nba)

        init = (tuple(jnp.full((16,), -1.0, jnp.float32) for _ in range(_T)),
                tuple(jnp.full((16,), 1.0, jnp.float32) for _ in range(_T)),
                tuple(jnp.zeros((16,), jnp.int32) for _ in range(_T)))
        bis, bus, bas = lax.fori_loop(0, _G, gt_body, init)
        for t in range(_T):
            bi_v[pl.ds(off + t * 16, 16)] = bis[t]
            bu_v[pl.ds(off + t * 16, 16)] = bus[t]
            ag_v[pl.ds(off + t * 16, 16)] = bas[t]
        return 0

    lax.fori_loop(0, _NV // _T, vec_body, 0)

    z16 = jnp.zeros((16,), jnp.int32)
    zf16 = jnp.zeros((16,), jnp.float32)
    for k in range(_RECI // 16):
        ri_v[pl.ds(k * 16, 16)] = z16

    def sel_body(v, carry):
        fgc, bgc = carry
        off = v * 16
        gidx = base + off + iota
        valid = gidx < _N
        bi = bi_v[pl.ds(off, 16)]
        bu = bu_v[pl.ds(off, 16)]
        ag = ag_v[pl.ds(off, 16)]
        i4 = (iota + off) * 4
        x1 = plsc.load_gather(rv_v, [i4])
        y1 = plsc.load_gather(rv_v, [i4 + 1])
        x2 = plsc.load_gather(rv_v, [i4 + 2])
        y2 = plsc.load_gather(rv_v, [i4 + 3])
        bi2 = bi + bi
        fgm = (bi2 > bu) & valid
        bgm = (bi2 < bu) & valid
        pf = plsc.cumsum(fgm.astype(jnp.int32)) - 1 + fgc
        mf = fgm & (pf < _MAX_POS)
        plsc.store_scatter(ri_v, [pf + 16], ag, mask=mf)
        plsc.store_scatter(rf_v, [pf], x1, mask=mf)
        plsc.store_scatter(rf_v, [pf + _MAX_POS], y1, mask=mf)
        plsc.store_scatter(rf_v, [pf + 2 * _MAX_POS], x2, mask=mf)
        plsc.store_scatter(rf_v, [pf + 3 * _MAX_POS], y2, mask=mf)
        pb = plsc.cumsum(bgm.astype(jnp.int32)) - 1 + bgc
        mb = bgm & (pb < _N_BG)
        plsc.store_scatter(rf_v, [pb + 128], x1, mask=mb)
        plsc.store_scatter(rf_v, [pb + 128 + _N_BG], y1, mask=mb)
        plsc.store_scatter(rf_v, [pb + 128 + 2 * _N_BG], x2, mask=mb)
        plsc.store_scatter(rf_v, [pb + 128 + 3 * _N_BG], y2, mask=mb)
        fgc = fgc + plsc.all_reduce_population_count(fgm)
        bgc = bgc + plsc.all_reduce_population_count(bgm)
        return fgc, bgc

    fgc, bgc = lax.fori_loop(0, _NV, sel_body, (z16, z16))
    fgc = jnp.minimum(fgc, _MAX_POS)
    bgc = jnp.minimum(bgc, _N_BG)
    arg0 = plsc.load_gather(ag_v, [z16])
    hdr = jnp.where(iota == 0, fgc,
                    jnp.where(iota == 1, bgc,
                              jnp.where(iota == 2, arg0, z16)))
    ri_v[pl.ds(0, 16)] = hdr
    r0 = plsc.load_gather(rv_v, [jnp.minimum(iota, 3)])
    rf_v[pl.ds(512, 16)] = jnp.where(iota < 4, r0, zf16)
    pltpu.sync_copy(ri_v, reci_hbm.at[pl.ds(wid * _RECI, _RECI)])
    pltpu.sync_copy(rf_v, recf_hbm.at[pl.ds(wid * _RECF, _RECF)])


@functools.partial(
    pl.kernel,
    out_type=[
        jax.ShapeDtypeStruct((_TOTAL, 4), jnp.float32),
        jax.ShapeDtypeStruct((_TOTAL,), jnp.int32),
        jax.ShapeDtypeStruct((_TOTAL * _OUTW,), jnp.float32),
        jax.ShapeDtypeStruct((_TOTAL * _OUTW,), jnp.float32),
    ],
    mesh=_mesh,
    compiler_params=_cparams,
    scratch_types=[
        pltpu.VMEM((_NW * _RECI,), jnp.int32),
        pltpu.VMEM((_NW * _RECF,), jnp.float32),
        pltpu.VMEM((_MAX_POS,), jnp.int32),
        pltpu.VMEM((_TOTAL, 4), jnp.float32),
        pltpu.VMEM((4 * _G,), jnp.float32),
        pltpu.VMEM((_G,), jnp.int32),
        pltpu.VMEM((_TOTAL,), jnp.int32),
        pltpu.VMEM((_TOTAL * _OUTW,), jnp.float32),
        pltpu.VMEM((_TOTAL * _OUTW,), jnp.float32),
    ],
)
def _phase2(reci_hbm, recf_hbm, gt_hbm, glab_hbm,
            orois_hbm, olab_hbm, otgt_hbm, oinw_hbm,
            ci_v, cf_v, fggt_v, rois_v, gtc_v, glab_v, lab_v, tgt_v, inw_v):
    wid = lax.axis_index("s") * 2 + lax.axis_index("c")

    @pl.when(wid == 0)
    def _():
        pltpu.sync_copy(gt_hbm, gtc_v)
        pltpu.sync_copy(glab_hbm, glab_v)
        iota = lax.iota(jnp.int32, 16)
        zf = jnp.zeros((16,), jnp.float32)
        z16 = jnp.zeros((16,), jnp.int32)
        pltpu.sync_copy(reci_hbm, ci_v)
        pltpu.sync_copy(recf_hbm, cf_v)
        arg0 = plsc.load_gather(ci_v, [z16 + 2])
        fggt_v[pl.ds(0, 16)] = arg0
        fggt_v[pl.ds(16, 16)] = arg0
        for c in range(4):
            r0c = plsc.load_gather(cf_v, [z16 + 512 + c])
            for s in range(_TOTAL // 16):
                plsc.store_scatter(rois_v, [iota + s * 16, z16 + c], r0c)

        def merge_body(ci, carry):
            fo, bo = carry
            ib = jnp.full((16,), ci * _RECI, dtype=jnp.int32)
            fb = jnp.full((16,), ci * _RECF, dtype=jnp.int32)
            cfgv = plsc.load_gather(ci_v, [ib])
            cbgv = plsc.load_gather(ci_v, [ib + 1])
            for k in range(_MAX_POS // 16):
                lpos = iota + k * 16
                dpos = fo + lpos
                m = (lpos < cfgv) & (dpos < _MAX_POS)
                gts = plsc.load_gather(ci_v, [ib + lpos + 16])
                plsc.store_scatter(fggt_v, [dpos], gts, mask=m)
                for c in range(4):
                    cv = plsc.load_gather(cf_v, [fb + lpos + c * _MAX_POS])
                    plsc.store_scatter(rois_v, [dpos, z16 + c], cv, mask=m)
            for k in range(_N_BG // 16):
                lpos = iota + k * 16
                dpos = bo + lpos
                m = (lpos < cbgv) & (dpos < _N_BG)
                for c in range(4):
                    cv = plsc.load_gather(cf_v, [fb + lpos + 128 + c * _N_BG])
                    plsc.store_scatter(rois_v, [dpos + _MAX_POS, z16 + c],
                                       cv, mask=m)
            return fo + cfgv, bo + cbgv

        lax.fori_loop(0, _NW, merge_body, (z16, z16))
        pltpu.sync_copy(rois_v, orois_hbm)

        def zero_body(i, _):
            for u in range(4):
                tgt_v[pl.ds((i * 4 + u) * 16, 16)] = zf
                inw_v[pl.ds((i * 4 + u) * 16, 16)] = zf
            return 0

        lax.fori_loop(0, _TOTAL * _OUTW // 64, zero_body, 0)

        one = jnp.ones((16,), jnp.float32)
        for k in range(_MAX_POS // 16):
            rows = iota + k * 16
            fgt = fggt_v[pl.ds(k * 16, 16)]
            fgt4 = fgt * 4
            lab = plsc.load_gather(glab_v, [fgt])
            lab_v[pl.ds(k * 16, 16)] = lab
            px1 = plsc.load_gather(rois_v, [rows, z16])
            py1 = plsc.load_gather(rois_v, [rows, z16 + 1])
            px2 = plsc.load_gather(rois_v, [rows, z16 + 2])
            py2 = plsc.load_gather(rois_v, [rows, z16 + 3])
            gx1 = plsc.load_gather(gtc_v, [fgt4])
            gy1 = plsc.load_gather(gtc_v, [fgt4 + 1])
            gx2 = plsc.load_gather(gtc_v, [fgt4 + 2])
            gy2 = plsc.load_gather(gtc_v, [fgt4 + 3])
            pw = px2 - px1 + 1.0
            ph = py2 - py1 + 1.0
            pxc = px1 + 0.5 * pw
            pyc = py1 + 0.5 * ph
            gw = gx2 - gx1 + 1.0
            gh = gy2 - gy1 + 1.0
            gxc = gx1 + 0.5 * gw
            gyc = gy1 + 0.5 * gh
            dx = (gxc - pxc) / pw
            dy = (gyc - pyc) / ph
            dw = _vlog(gw / pw)
            dh = _vlog(gh / ph)
            flat = rows * _OUTW + lab * 4
            plsc.store_scatter(tgt_v, [flat], dx)
            plsc.store_scatter(tgt_v, [flat + 1], dy)
            plsc.store_scatter(tgt_v, [flat + 2], dw)
            plsc.store_scatter(tgt_v, [flat + 3], dh)
            plsc.store_scatter(inw_v, [flat], one)
            plsc.store_scatter(inw_v, [flat + 1], one)
            plsc.store_scatter(inw_v, [flat + 2], one)
            plsc.store_scatter(inw_v, [flat + 3], one)

        for k in range(2, _TOTAL // 16):
            lab_v[pl.ds(k * 16, 16)] = z16
        pltpu.sync_copy(lab_v, olab_hbm)
        pltpu.sync_copy(tgt_v, otgt_hbm)
        pltpu.sync_copy(inw_v, oinw_hbm)


def kernel(rois, gt_bboxes, gt_labels):
    reci, recf = _phase1(rois.reshape(-1), gt_bboxes.reshape(-1))
    frois, flab, ftgt, finw = _phase2(
        reci, recf, gt_bboxes.reshape(-1), gt_labels.astype(jnp.int32))
    ftgt = ftgt.reshape(_TOTAL, _OUTW)
    finw = finw.reshape(_TOTAL, _OUTW)
    foutw = jnp.ones((_TOTAL, _OUTW), jnp.float32)
    return frois, flab, ftgt, finw, foutw


# speedup vs baseline: 1.2185x; 1.2185x over previous
import functools

import jax
import jax.numpy as jnp
from jax import lax
from jax.experimental import pallas as pl
from jax.experimental.pallas import tpu as pltpu
from jax.experimental.pallas import tpu_sc as plsc

_NUM_CLASSES = 21
_N = 20000
_NPAD = 20480
_CHUNK = 640
_NV = _CHUNK // 16
_G = 64
_MAX_POS = 32
_N_BG = 96
_TOTAL = 128
_RECI = 64
_RECF = 528
_NW = 32

_mesh = plsc.VectorSubcoreMesh(core_axis_name="c", subcore_axis_name="s",
                               num_cores=2)
_cparams = pltpu.CompilerParams(needs_layout_passes=False)


def _vlog(x):
    bits = plsc.bitcast(x, jnp.int32)
    e = ((bits >> 23) & 0xFF) - 127
    m = plsc.bitcast((bits & 0x7FFFFF) | 0x3F800000, jnp.float32)
    big = m > 1.4142135623730951
    m = jnp.where(big, m * 0.5, m)
    e = jnp.where(big, e + 1, e)
    r = (m - 1.0) / (m + 1.0)
    r2 = r * r
    p = 1.0 / 7.0 + r2 * (1.0 / 9.0)
    p = 1.0 / 5.0 + r2 * p
    p = 1.0 / 3.0 + r2 * p
    p = 1.0 + r2 * p
    return e.astype(jnp.float32) * 0.6931471805599453 + 2.0 * r * p


@functools.partial(
    pl.kernel,
    out_type=[
        jax.ShapeDtypeStruct((_NW * _RECI,), jnp.int32),
        jax.ShapeDtypeStruct((_NW * _RECF,), jnp.float32),
    ],
    mesh=_mesh,
    compiler_params=_cparams,
    scratch_types=[
        pltpu.VMEM((_CHUNK,), jnp.float32),
        pltpu.VMEM((_CHUNK,), jnp.float32),
        pltpu.VMEM((_CHUNK,), jnp.float32),
        pltpu.VMEM((_CHUNK,), jnp.float32),
        pltpu.VMEM((4 * _G,), jnp.float32),
        pltpu.VMEM((_G,), jnp.float32),
        pltpu.VMEM((_CHUNK,), jnp.float32),
        pltpu.VMEM((_CHUNK,), jnp.float32),
        pltpu.VMEM((_CHUNK,), jnp.int32),
        pltpu.VMEM((_RECI,), jnp.int32),
        pltpu.VMEM((_RECF,), jnp.float32),
    ],
)
def _phase1(rx1_hbm, ry1_hbm, rx2_hbm, ry2_hbm, gt_hbm, reci_hbm, recf_hbm,
            x1_v, y1_v, x2_v, y2_v, gt_v, a2_v, bi_v, bu_v, ag_v, ri_v, rf_v):
    wid = lax.axis_index("s") * 2 + lax.axis_index("c")
    base = wid * _CHUNK
    pltpu.sync_copy(rx1_hbm.at[pl.ds(base, _CHUNK)], x1_v)
    pltpu.sync_copy(ry1_hbm.at[pl.ds(base, _CHUNK)], y1_v)
    pltpu.sync_copy(rx2_hbm.at[pl.ds(base, _CHUNK)], x2_v)
    pltpu.sync_copy(ry2_hbm.at[pl.ds(base, _CHUNK)], y2_v)
    pltpu.sync_copy(gt_hbm, gt_v)

    iota = lax.iota(jnp.int32, 16)
    for k in range(_G // 16):
        gx1 = gt_v[pl.ds(k * 16, 16)]
        gy1 = gt_v[pl.ds(_G + k * 16, 16)]
        gx2 = gt_v[pl.ds(2 * _G + k * 16, 16)]
        gy2 = gt_v[pl.ds(3 * _G + k * 16, 16)]
        a2_v[pl.ds(k * 16, 16)] = (gx2 - gx1) * (gy2 - gy1)

    _T = 4

    def vec_body(q, _):
        off = q * (16 * _T)
        x1s = [x1_v[pl.ds(off + t * 16, 16)] for t in range(_T)]
        y1s = [y1_v[pl.ds(off + t * 16, 16)] for t in range(_T)]
        x2s = [x2_v[pl.ds(off + t * 16, 16)] for t in range(_T)]
        y2s = [y2_v[pl.ds(off + t * 16, 16)] for t in range(_T)]
        a1s = [(x2s[t] - x1s[t]) * (y2s[t] - y1s[t]) for t in range(_T)]

        def gt_body(j, carry):
            bis, bus, bas = carry
            js = jnp.full((16,), j, dtype=jnp.int32)
            gx1 = plsc.load_gather(gt_v, [js])
            gy1 = plsc.load_gather(gt_v, [js + _G])
            gx2 = plsc.load_gather(gt_v, [js + 2 * _G])
            gy2 = plsc.load_gather(gt_v, [js + 3 * _G])
            a2 = plsc.load_gather(a2_v, [js])
            nbi, nbu, nba = [], [], []
            for t in range(_T):
                w = jnp.minimum(x2s[t], gx2) - jnp.maximum(x1s[t], gx1)
                h = jnp.minimum(y2s[t], gy2) - jnp.maximum(y1s[t], gy1)
                inter = jnp.maximum(w, 0.0) * jnp.maximum(h, 0.0)
                union = jnp.maximum(a1s[t] + a2 - inter, 1e-8)
                better = inter * bus[t] > bis[t] * union
                nbi.append(jnp.where(better, inter, bis[t]))
                nbu.append(jnp.where(better, union, bus[t]))
                nba.append(jnp.where(better, js, bas[t]))
            return tuple(nbi), tuple(nbu), tuple(nba)

        init = (tuple(jnp.full((16,), -1.0, jnp.float32) for _ in range(_T)),
                tuple(jnp.full((16,), 1.0, jnp.float32) for _ in range(_T)),
                tuple(jnp.zeros((16,), jnp.int32) for _ in range(_T)))
        bis, bus, bas = lax.fori_loop(0, _G, gt_body, init)
        for t in range(_T):
            bi_v[pl.ds(off + t * 16, 16)] = bis[t]
            bu_v[pl.ds(off + t * 16, 16)] = bus[t]
            ag_v[pl.ds(off + t * 16, 16)] = bas[t]
        return 0

    lax.fori_loop(0, _NV // _T, vec_body, 0)

    z16 = jnp.zeros((16,), jnp.int32)
    zf16 = jnp.zeros((16,), jnp.float32)
    for k in range(_RECI // 16):
        ri_v[pl.ds(k * 16, 16)] = z16

    def sel_body(v, carry):
        fgc, bgc = carry
        off = v * 16
        gidx = base + off + iota
        valid = gidx < _N
        bi = bi_v[pl.ds(off, 16)]
        bu = bu_v[pl.ds(off, 16)]
        ag = ag_v[pl.ds(off, 16)]
        x1 = x1_v[pl.ds(off, 16)]
        y1 = y1_v[pl.ds(off, 16)]
        x2 = x2_v[pl.ds(off, 16)]
        y2 = y2_v[pl.ds(off, 16)]
        bi2 = bi + bi
        fgm = (bi2 > bu) & valid
        bgm = (bi2 < bu) & valid
        pf = plsc.cumsum(fgm.astype(jnp.int32)) - 1 + fgc
        mf = fgm & (pf < _MAX_POS)
        plsc.store_scatter(ri_v, [pf + 16], ag, mask=mf)
        plsc.store_scatter(rf_v, [pf], x1, mask=mf)
        plsc.store_scatter(rf_v, [pf + _MAX_POS], y1, mask=mf)
        plsc.store_scatter(rf_v, [pf + 2 * _MAX_POS], x2, mask=mf)
        plsc.store_scatter(rf_v, [pf + 3 * _MAX_POS], y2, mask=mf)
        pb = plsc.cumsum(bgm.astype(jnp.int32)) - 1 + bgc
        mb = bgm & (pb < _N_BG)
        plsc.store_scatter(rf_v, [pb + 128], x1, mask=mb)
        plsc.store_scatter(rf_v, [pb + 128 + _N_BG], y1, mask=mb)
        plsc.store_scatter(rf_v, [pb + 128 + 2 * _N_BG], x2, mask=mb)
        plsc.store_scatter(rf_v, [pb + 128 + 3 * _N_BG], y2, mask=mb)
        fgc = fgc + plsc.all_reduce_population_count(fgm)
        bgc = bgc + plsc.all_reduce_population_count(bgm)
        return fgc, bgc

    fgc, bgc = lax.fori_loop(0, _NV, sel_body, (z16, z16))
    fgc = jnp.minimum(fgc, _MAX_POS)
    bgc = jnp.minimum(bgc, _N_BG)
    arg0 = plsc.load_gather(ag_v, [z16])
    hdr = jnp.where(iota == 0, fgc,
                    jnp.where(iota == 1, bgc,
                              jnp.where(iota == 2, arg0, z16)))
    ri_v[pl.ds(0, 16)] = hdr
    x10 = plsc.load_gather(x1_v, [z16])
    y10 = plsc.load_gather(y1_v, [z16])
    x20 = plsc.load_gather(x2_v, [z16])
    y20 = plsc.load_gather(y2_v, [z16])
    hdrf = jnp.where(iota == 0, x10,
                     jnp.where(iota == 1, y10,
                               jnp.where(iota == 2, x20,
                                         jnp.where(iota == 3, y20, zf16))))
    rf_v[pl.ds(512, 16)] = hdrf
    pltpu.sync_copy(ri_v, reci_hbm.at[pl.ds(wid * _RECI, _RECI)])
    pltpu.sync_copy(rf_v, recf_hbm.at[pl.ds(wid * _RECF, _RECF)])


@functools.partial(
    pl.kernel,
    out_type=[
        jax.ShapeDtypeStruct((_TOTAL, 4), jnp.float32),
        jax.ShapeDtypeStruct((_TOTAL,), jnp.int32),
        jax.ShapeDtypeStruct((_TOTAL * _NUM_CLASSES * 4,), jnp.float32),
        jax.ShapeDtypeStruct((_TOTAL * _NUM_CLASSES * 4,), jnp.float32),
    ],
    mesh=_mesh,
    compiler_params=_cparams,
    scratch_types=[
        pltpu.VMEM((_NW * _RECI,), jnp.int32),
        pltpu.VMEM((_NW * _RECF,), jnp.float32),
        pltpu.VMEM((_MAX_POS,), jnp.int32),
        pltpu.VMEM((_TOTAL, 4), jnp.float32),
        pltpu.VMEM((4 * _G,), jnp.float32),
        pltpu.VMEM((_G,), jnp.int32),
        pltpu.VMEM((_TOTAL,), jnp.int32),
        pltpu.VMEM((_TOTAL * _NUM_CLASSES * 4,), jnp.float32),
        pltpu.VMEM((_TOTAL * _NUM_CLASSES * 4,), jnp.float32),
    ],
)
def _phase2(reci_hbm, recf_hbm, gt_hbm, glab_hbm,
            orois_hbm, olab_hbm, otgt_hbm, oinw_hbm,
            ai_v, af_v, fggt_v, rois_v, gtc_v, glab_v, lab_v, tgt_v, inw_v):
    wid = lax.axis_index("s") * 2 + lax.axis_index("c")

    @pl.when(wid == 0)
    def _():
        pltpu.sync_copy(reci_hbm, ai_v)
        pltpu.sync_copy(recf_hbm, af_v)
        pltpu.sync_copy(gt_hbm, gtc_v)
        pltpu.sync_copy(glab_hbm, glab_v)
        iota = lax.iota(jnp.int32, 16)
        z16 = jnp.zeros((16,), jnp.int32)

        arg0 = plsc.load_gather(ai_v, [z16 + 2])
        fggt_v[pl.ds(0, 16)] = arg0
        fggt_v[pl.ds(16, 16)] = arg0
        for c in range(4):
            r0c = plsc.load_gather(af_v, [z16 + 512 + c])
            for s in range(_TOTAL // 16):
                plsc.store_scatter(rois_v, [iota + s * 16, z16 + c], r0c)

        def chunk_body(ci, carry):
            fo, bo = carry
            ib = jnp.full((16,), ci * _RECI, dtype=jnp.int32)
            fb = jnp.full((16,), ci * _RECF, dtype=jnp.int32)
            cfg = plsc.load_gather(ai_v, [ib])
            cbg = plsc.load_gather(ai_v, [ib + 1])
            for k in range(_MAX_POS // 16):
                lpos = iota + k * 16
                dpos = fo + lpos
                m = (lpos < cfg) & (dpos < _MAX_POS)
                gts = plsc.load_gather(ai_v, [ib + 16 + lpos])
                plsc.store_scatter(fggt_v, [dpos], gts, mask=m)
                for c in range(4):
                    cv = plsc.load_gather(af_v, [fb + lpos + c * _MAX_POS])
                    plsc.store_scatter(rois_v, [dpos, z16 + c], cv, mask=m)
            for k in range(_N_BG // 16):
                lpos = iota + k * 16
                dpos = bo + lpos
                m = (lpos < cbg) & (dpos < _N_BG)
                for c in range(4):
                    cv = plsc.load_gather(af_v, [fb + lpos + 128 + c * _N_BG])
                    plsc.store_scatter(rois_v, [dpos + _MAX_POS, z16 + c],
                                       cv, mask=m)
            return fo + cfg, bo + cbg

        lax.fori_loop(0, _NW, chunk_body, (z16, z16))
        pltpu.sync_copy(rois_v, orois_hbm)

        zf = jnp.zeros((16,), jnp.float32)

        def zero_body(i, _):
            tgt_v[pl.ds(i * 16, 16)] = zf
            inw_v[pl.ds(i * 16, 16)] = zf
            return 0

        lax.fori_loop(0, _TOTAL * _NUM_CLASSES * 4 // 16, zero_body, 0)

        one = jnp.ones((16,), jnp.float32)
        for k in range(_MAX_POS // 16):
            rows = iota + k * 16
            fgt = fggt_v[pl.ds(k * 16, 16)]
            lab = plsc.load_gather(glab_v, [fgt])
            lab_v[pl.ds(k * 16, 16)] = lab
            px1 = plsc.load_gather(rois_v, [rows, z16])
            py1 = plsc.load_gather(rois_v, [rows, z16 + 1])
            px2 = plsc.load_gather(rois_v, [rows, z16 + 2])
            py2 = plsc.load_gather(rois_v, [rows, z16 + 3])
            gx1 = plsc.load_gather(gtc_v, [fgt])
            gy1 = plsc.load_gather(gtc_v, [fgt + _G])
            gx2 = plsc.load_gather(gtc_v, [fgt + 2 * _G])
            gy2 = plsc.load_gather(gtc_v, [fgt + 3 * _G])
            pw = px2 - px1 + 1.0
            ph = py2 - py1 + 1.0
            pxc = px1 + 0.5 * pw
            pyc = py1 + 0.5 * ph
            gw = gx2 - gx1 + 1.0
            gh = gy2 - gy1 + 1.0
            gxc = gx1 + 0.5 * gw
            gyc = gy1 + 0.5 * gh
            dx = (gxc - pxc) / pw
            dy = (gyc - pyc) / ph
            dw = _vlog(gw / pw)
            dh = _vlog(gh / ph)
            flat = rows * (_NUM_CLASSES * 4) + lab * 4
            plsc.store_scatter(tgt_v, [flat], dx)
            plsc.store_scatter(tgt_v, [flat + 1], dy)
            plsc.store_scatter(tgt_v, [flat + 2], dw)
            plsc.store_scatter(tgt_v, [flat + 3], dh)
            plsc.store_scatter(inw_v, [flat], one)
            plsc.store_scatter(inw_v, [flat + 1], one)
            plsc.store_scatter(inw_v, [flat + 2], one)
            plsc.store_scatter(inw_v, [flat + 3], one)

        for k in range(2, _TOTAL // 16):
            lab_v[pl.ds(k * 16, 16)] = z16
        pltpu.sync_copy(lab_v, olab_hbm)
        pltpu.sync_copy(tgt_v, otgt_hbm)
        pltpu.sync_copy(inw_v, oinw_hbm)


def kernel(rois, gt_bboxes, gt_labels):
    rois_pad = jnp.pad(rois, ((0, _NPAD - _N), (0, 0)))
    gt_cols = gt_bboxes.T.reshape(-1)
    reci, recf = _phase1(rois_pad[:, 0], rois_pad[:, 1],
                         rois_pad[:, 2], rois_pad[:, 3], gt_cols)
    frois, flab, ftgt, finw = _phase2(
        reci, recf, gt_cols, gt_labels.astype(jnp.int32))
    ftgt = ftgt.reshape(_TOTAL, _NUM_CLASSES * 4)
    finw = finw.reshape(_TOTAL, _NUM_CLASSES * 4)
    foutw = jnp.ones((_TOTAL, _NUM_CLASSES * 4), jnp.float32)
    return frois, flab, ftgt, finw, foutw


# speedup vs baseline: 1.3320x; 1.0931x over previous
import functools

import jax
import jax.numpy as jnp
from jax import lax
from jax.experimental import pallas as pl
from jax.experimental.pallas import tpu as pltpu
from jax.experimental.pallas import tpu_sc as plsc

_NUM_CLASSES = 21
_N = 20000
_NPAD = 20480
_CHUNK = 640
_NV = _CHUNK // 16
_G = 64
_MAX_POS = 32
_N_BG = 96
_TOTAL = 128
_RECI = 64
_RECF = 528
_NW = 32

_mesh = plsc.VectorSubcoreMesh(core_axis_name="c", subcore_axis_name="s",
                               num_cores=2)
_cparams = pltpu.CompilerParams(needs_layout_passes=False)


def _vlog(x):
    bits = plsc.bitcast(x, jnp.int32)
    e = ((bits >> 23) & 0xFF) - 127
    m = plsc.bitcast((bits & 0x7FFFFF) | 0x3F800000, jnp.float32)
    big = m > 1.4142135623730951
    m = jnp.where(big, m * 0.5, m)
    e = jnp.where(big, e + 1, e)
    r = (m - 1.0) / (m + 1.0)
    r2 = r * r
    p = 1.0 / 7.0 + r2 * (1.0 / 9.0)
    p = 1.0 / 5.0 + r2 * p
    p = 1.0 / 3.0 + r2 * p
    p = 1.0 + r2 * p
    return e.astype(jnp.float32) * 0.6931471805599453 + 2.0 * r * p


@functools.partial(
    pl.kernel,
    out_type=[
        jax.ShapeDtypeStruct((_NW * _RECI,), jnp.int32),
        jax.ShapeDtypeStruct((_NW * _RECF,), jnp.float32),
    ],
    mesh=_mesh,
    compiler_params=_cparams,
    scratch_types=[
        pltpu.VMEM((_CHUNK,), jnp.float32),
        pltpu.VMEM((_CHUNK,), jnp.float32),
        pltpu.VMEM((_CHUNK,), jnp.float32),
        pltpu.VMEM((_CHUNK,), jnp.float32),
        pltpu.VMEM((4 * _G,), jnp.float32),
        pltpu.VMEM((_G,), jnp.float32),
        pltpu.VMEM((_CHUNK,), jnp.float32),
        pltpu.VMEM((_CHUNK,), jnp.float32),
        pltpu.VMEM((_CHUNK,), jnp.int32),
        pltpu.VMEM((_RECI,), jnp.int32),
        pltpu.VMEM((_RECF,), jnp.float32),
    ],
)
def _phase1(rx1_hbm, ry1_hbm, rx2_hbm, ry2_hbm, gt_hbm, reci_hbm, recf_hbm,
            x1_v, y1_v, x2_v, y2_v, gt_v, a2_v, bi_v, bu_v, ag_v, ri_v, rf_v):
    wid = lax.axis_index("s") * 2 + lax.axis_index("c")
    base = wid * _CHUNK
    pltpu.sync_copy(rx1_hbm.at[pl.ds(base, _CHUNK)], x1_v)
    pltpu.sync_copy(ry1_hbm.at[pl.ds(base, _CHUNK)], y1_v)
    pltpu.sync_copy(rx2_hbm.at[pl.ds(base, _CHUNK)], x2_v)
    pltpu.sync_copy(ry2_hbm.at[pl.ds(base, _CHUNK)], y2_v)
    pltpu.sync_copy(gt_hbm, gt_v)

    iota = lax.iota(jnp.int32, 16)
    for k in range(_G // 16):
        gx1 = gt_v[pl.ds(k * 16, 16)]
        gy1 = gt_v[pl.ds(_G + k * 16, 16)]
        gx2 = gt_v[pl.ds(2 * _G + k * 16, 16)]
        gy2 = gt_v[pl.ds(3 * _G + k * 16, 16)]
        a2_v[pl.ds(k * 16, 16)] = (gx2 - gx1) * (gy2 - gy1)

    _T = 4

    def vec_body(q, _):
        off = q * (16 * _T)
        x1s = [x1_v[pl.ds(off + t * 16, 16)] for t in range(_T)]
        y1s = [y1_v[pl.ds(off + t * 16, 16)] for t in range(_T)]
        x2s = [x2_v[pl.ds(off + t * 16, 16)] for t in range(_T)]
        y2s = [y2_v[pl.ds(off + t * 16, 16)] for t in range(_T)]
        a1s = [(x2s[t] - x1s[t]) * (y2s[t] - y1s[t]) for t in range(_T)]

        def gt_body(j, carry):
            bis, bus, bas = carry
            js = jnp.full((16,), j, dtype=jnp.int32)
            gx1 = plsc.load_gather(gt_v, [js])
            gy1 = plsc.load_gather(gt_v, [js + _G])
            gx2 = plsc.load_gather(gt_v, [js + 2 * _G])
            gy2 = plsc.load_gather(gt_v, [js + 3 * _G])
            a2 = plsc.load_gather(a2_v, [js])
            nbi, nbu, nba = [], [], []
            for t in range(_T):
                w = jnp.minimum(x2s[t], gx2) - jnp.maximum(x1s[t], gx1)
                h = jnp.minimum(y2s[t], gy2) - jnp.maximum(y1s[t], gy1)
                inter = jnp.maximum(w, 0.0) * jnp.maximum(h, 0.0)
                union = jnp.maximum(a1s[t] + a2 - inter, 1e-8)
                better = inter * bus[t] > bis[t] * union
                nbi.append(jnp.where(better, inter, bis[t]))
                nbu.append(jnp.where(better, union, bus[t]))
                nba.append(jnp.where(better, js, bas[t]))
            return tuple(nbi), tuple(nbu), tuple(nba)

        init = (tuple(jnp.full((16,), -1.0, jnp.float32) for _ in range(_T)),
                tuple(jnp.full((16,), 1.0, jnp.float32) for _ in range(_T)),
                tuple(jnp.zeros((16,), jnp.int32) for _ in range(_T)))
        bis, bus, bas = lax.fori_loop(0, _G, gt_body, init)
        for t in range(_T):
            bi_v[pl.ds(off + t * 16, 16)] = bis[t]
            bu_v[pl.ds(off + t * 16, 16)] = bus[t]
            ag_v[pl.ds(off + t * 16, 16)] = bas[t]
        return 0

    lax.fori_loop(0, _NV // _T, vec_body, 0)

    z16 = jnp.zeros((16,), jnp.int32)
    zf16 = jnp.zeros((16,), jnp.float32)
    for k in range(_RECI // 16):
        ri_v[pl.ds(k * 16, 16)] = z16

    def sel_body(v, carry):
        fgc, bgc = carry
        off = v * 16
        gidx = base + off + iota
        valid = gidx < _N
        bi = bi_v[pl.ds(off, 16)]
        bu = bu_v[pl.ds(off, 16)]
        ag = ag_v[pl.ds(off, 16)]
        x1 = x1_v[pl.ds(off, 16)]
        y1 = y1_v[pl.ds(off, 16)]
        x2 = x2_v[pl.ds(off, 16)]
        y2 = y2_v[pl.ds(off, 16)]
        bi2 = bi + bi
        fgm = (bi2 > bu) & valid
        bgm = (bi2 < bu) & valid
        pf = plsc.cumsum(fgm.astype(jnp.int32)) - 1 + fgc
        mf = fgm & (pf < _MAX_POS)
        plsc.store_scatter(ri_v, [pf + 16], ag, mask=mf)
        plsc.store_scatter(rf_v, [pf], x1, mask=mf)
        plsc.store_scatter(rf_v, [pf + _MAX_POS], y1, mask=mf)
        plsc.store_scatter(rf_v, [pf + 2 * _MAX_POS], x2, mask=mf)
        plsc.store_scatter(rf_v, [pf + 3 * _MAX_POS], y2, mask=mf)
        pb = plsc.cumsum(bgm.astype(jnp.int32)) - 1 + bgc
        mb = bgm & (pb < _N_BG)
        plsc.store_scatter(rf_v, [pb + 128], x1, mask=mb)
        plsc.store_scatter(rf_v, [pb + 128 + _N_BG], y1, mask=mb)
        plsc.store_scatter(rf_v, [pb + 128 + 2 * _N_BG], x2, mask=mb)
        plsc.store_scatter(rf_v, [pb + 128 + 3 * _N_BG], y2, mask=mb)
        fgc = fgc + plsc.all_reduce_population_count(fgm)
        bgc = bgc + plsc.all_reduce_population_count(bgm)
        return fgc, bgc

    fgc, bgc = lax.fori_loop(0, _NV, sel_body, (z16, z16))
    fgc = jnp.minimum(fgc, _MAX_POS)
    bgc = jnp.minimum(bgc, _N_BG)
    arg0 = plsc.load_gather(ag_v, [z16])
    hdr = jnp.where(iota == 0, fgc,
                    jnp.where(iota == 1, bgc,
                              jnp.where(iota == 2, arg0, z16)))
    ri_v[pl.ds(0, 16)] = hdr
    x10 = plsc.load_gather(x1_v, [z16])
    y10 = plsc.load_gather(y1_v, [z16])
    x20 = plsc.load_gather(x2_v, [z16])
    y20 = plsc.load_gather(y2_v, [z16])
    hdrf = jnp.where(iota == 0, x10,
                     jnp.where(iota == 1, y10,
                               jnp.where(iota == 2, x20,
                                         jnp.where(iota == 3, y20, zf16))))
    rf_v[pl.ds(512, 16)] = hdrf
    pltpu.sync_copy(ri_v, reci_hbm.at[pl.ds(wid * _RECI, _RECI)])
    pltpu.sync_copy(rf_v, recf_hbm.at[pl.ds(wid * _RECF, _RECF)])


@functools.partial(
    pl.kernel,
    out_type=[
        jax.ShapeDtypeStruct((_TOTAL, 4), jnp.float32),
        jax.ShapeDtypeStruct((_TOTAL,), jnp.int32),
        jax.ShapeDtypeStruct((_TOTAL * _NUM_CLASSES * 4,), jnp.float32),
        jax.ShapeDtypeStruct((_TOTAL * _NUM_CLASSES * 4,), jnp.float32),
    ],
    mesh=_mesh,
    compiler_params=_cparams,
    scratch_types=[
        pltpu.VMEM((_RECI,), jnp.int32),
        pltpu.VMEM((_RECF,), jnp.float32),
        pltpu.VMEM((_MAX_POS,), jnp.int32),
        pltpu.VMEM((_TOTAL, 4), jnp.float32),
        pltpu.VMEM((4 * _G,), jnp.float32),
        pltpu.VMEM((_G,), jnp.int32),
        pltpu.VMEM((_TOTAL,), jnp.int32),
        pltpu.VMEM((_TOTAL * _NUM_CLASSES * 4,), jnp.float32),
        pltpu.VMEM((_TOTAL * _NUM_CLASSES * 4,), jnp.float32),
    ],
)
def _phase2(reci_hbm, recf_hbm, gt_hbm, glab_hbm,
            orois_hbm, olab_hbm, otgt_hbm, oinw_hbm,
            ci_v, cf_v, fggt_v, rois_v, gtc_v, glab_v, lab_v, tgt_v, inw_v):
    wid = lax.axis_index("s") * 2 + lax.axis_index("c")

    @pl.when(wid == 0)
    def _():
        pltpu.sync_copy(gt_hbm, gtc_v)
        pltpu.sync_copy(glab_hbm, glab_v)
        iota = lax.iota(jnp.int32, 16)
        z16 = jnp.zeros((16,), jnp.int32)

        pltpu.sync_copy(reci_hbm.at[pl.ds(0, _RECI)], ci_v)
        pltpu.sync_copy(recf_hbm.at[pl.ds(0, _RECF)], cf_v)
        arg0 = plsc.load_gather(ci_v, [z16 + 2])
        fggt_v[pl.ds(0, 16)] = arg0
        fggt_v[pl.ds(16, 16)] = arg0
        for c in range(4):
            r0c = plsc.load_gather(cf_v, [z16 + 512 + c])
            for s in range(_TOTAL // 16):
                plsc.store_scatter(rois_v, [iota + s * 16, z16 + c], r0c)

        def merge_cond(carry):
            ci, fo, bo = carry
            return (ci < _NW) & ((fo < _MAX_POS) | (bo < _N_BG))

        def merge_body(carry):
            ci, fo, bo = carry
            pltpu.sync_copy(reci_hbm.at[pl.ds(ci * _RECI, _RECI)], ci_v)
            pltpu.sync_copy(recf_hbm.at[pl.ds(ci * _RECF, _RECF)], cf_v)
            hv = ci_v[pl.ds(0, 16)]
            cfg = hv[0]
            cbg = hv[1]
            cfgv = jnp.full((16,), cfg, dtype=jnp.int32)
            cbgv = jnp.full((16,), cbg, dtype=jnp.int32)
            fov = jnp.full((16,), fo, dtype=jnp.int32)
            bov = jnp.full((16,), bo, dtype=jnp.int32)
            for k in range(_MAX_POS // 16):
                lpos = iota + k * 16
                dpos = fov + lpos
                m = (lpos < cfgv) & (dpos < _MAX_POS)
                gts = plsc.load_gather(ci_v, [16 + lpos])
                plsc.store_scatter(fggt_v, [dpos], gts, mask=m)
                for c in range(4):
                    cv = plsc.load_gather(cf_v, [lpos + c * _MAX_POS])
                    plsc.store_scatter(rois_v, [dpos, z16 + c], cv, mask=m)
            for k in range(_N_BG // 16):
                lpos = iota + k * 16
                dpos = bov + lpos
                m = (lpos < cbgv) & (dpos < _N_BG)
                for c in range(4):
                    cv = plsc.load_gather(cf_v, [lpos + 128 + c * _N_BG])
                    plsc.store_scatter(rois_v, [dpos + _MAX_POS, z16 + c],
                                       cv, mask=m)
            return ci + 1, fo + cfg, bo + cbg

        lax.while_loop(merge_cond, merge_body, (0, 0, 0))
        pltpu.sync_copy(rois_v, orois_hbm)

        zf = jnp.zeros((16,), jnp.float32)

        def zero_body(i, _):
            tgt_v[pl.ds(i * 16, 16)] = zf
            inw_v[pl.ds(i * 16, 16)] = zf
            return 0

        lax.fori_loop(0, _TOTAL * _NUM_CLASSES * 4 // 16, zero_body, 0)

        one = jnp.ones((16,), jnp.float32)
        for k in range(_MAX_POS // 16):
            rows = iota + k * 16
            fgt = fggt_v[pl.ds(k * 16, 16)]
            lab = plsc.load_gather(glab_v, [fgt])
            lab_v[pl.ds(k * 16, 16)] = lab
            px1 = plsc.load_gather(rois_v, [rows, z16])
            py1 = plsc.load_gather(rois_v, [rows, z16 + 1])
            px2 = plsc.load_gather(rois_v, [rows, z16 + 2])
            py2 = plsc.load_gather(rois_v, [rows, z16 + 3])
            gx1 = plsc.load_gather(gtc_v, [fgt])
            gy1 = plsc.load_gather(gtc_v, [fgt + _G])
            gx2 = plsc.load_gather(gtc_v, [fgt + 2 * _G])
            gy2 = plsc.load_gather(gtc_v, [fgt + 3 * _G])
            pw = px2 - px1 + 1.0
            ph = py2 - py1 + 1.0
            pxc = px1 + 0.5 * pw
            pyc = py1 + 0.5 * ph
            gw = gx2 - gx1 + 1.0
            gh = gy2 - gy1 + 1.0
            gxc = gx1 + 0.5 * gw
            gyc = gy1 + 0.5 * gh
            dx = (gxc - pxc) / pw
            dy = (gyc - pyc) / ph
            dw = _vlog(gw / pw)
            dh = _vlog(gh / ph)
            flat = rows * (_NUM_CLASSES * 4) + lab * 4
            plsc.store_scatter(tgt_v, [flat], dx)
            plsc.store_scatter(tgt_v, [flat + 1], dy)
            plsc.store_scatter(tgt_v, [flat + 2], dw)
            plsc.store_scatter(tgt_v, [flat + 3], dh)
            plsc.store_scatter(inw_v, [flat], one)
            plsc.store_scatter(inw_v, [flat + 1], one)
            plsc.store_scatter(inw_v, [flat + 2], one)
            plsc.store_scatter(inw_v, [flat + 3], one)

        for k in range(2, _TOTAL // 16):
            lab_v[pl.ds(k * 16, 16)] = z16
        pltpu.sync_copy(lab_v, olab_hbm)
        pltpu.sync_copy(tgt_v, otgt_hbm)
        pltpu.sync_copy(inw_v, oinw_hbm)


def kernel(rois, gt_bboxes, gt_labels):
    rois_pad = jnp.pad(rois, ((0, _NPAD - _N), (0, 0)))
    gt_cols = gt_bboxes.T.reshape(-1)
    reci, recf = _phase1(rois_pad[:, 0], rois_pad[:, 1],
                         rois_pad[:, 2], rois_pad[:, 3], gt_cols)
    frois, flab, ftgt, finw = _phase2(
        reci, recf, gt_cols, gt_labels.astype(jnp.int32))
    ftgt = ftgt.reshape(_TOTAL, _NUM_CLASSES * 4)
    finw = finw.reshape(_TOTAL, _NUM_CLASSES * 4)
    foutw = jnp.ones((_TOTAL, _NUM_CLASSES * 4), jnp.float32)
    return frois, flab, ftgt, finw, foutw


# speedup vs baseline: 1.3961x; 1.0481x over previous
import functools

import jax
import jax.numpy as jnp
from jax import lax
from jax.experimental import pallas as pl
from jax.experimental.pallas import tpu as pltpu
from jax.experimental.pallas import tpu_sc as plsc

_NUM_CLASSES = 21
_N = 20000
_NPAD = 20480
_CHUNK = 640
_NV = _CHUNK // 16
_G = 64
_MAX_POS = 32
_N_BG = 96
_TOTAL = 128
_RECI = 64
_RECF = 528
_NW = 32

_mesh = plsc.VectorSubcoreMesh(core_axis_name="c", subcore_axis_name="s",
                               num_cores=2)
_cparams = pltpu.CompilerParams(needs_layout_passes=False)


def _vlog(x):
    bits = plsc.bitcast(x, jnp.int32)
    e = ((bits >> 23) & 0xFF) - 127
    m = plsc.bitcast((bits & 0x7FFFFF) | 0x3F800000, jnp.float32)
    big = m > 1.4142135623730951
    m = jnp.where(big, m * 0.5, m)
    e = jnp.where(big, e + 1, e)
    r = (m - 1.0) / (m + 1.0)
    r2 = r * r
    p = 1.0 / 7.0 + r2 * (1.0 / 9.0)
    p = 1.0 / 5.0 + r2 * p
    p = 1.0 / 3.0 + r2 * p
    p = 1.0 + r2 * p
    return e.astype(jnp.float32) * 0.6931471805599453 + 2.0 * r * p


@functools.partial(
    pl.kernel,
    out_type=[
        jax.ShapeDtypeStruct((_NW * _RECI,), jnp.int32),
        jax.ShapeDtypeStruct((_NW * _RECF,), jnp.float32),
    ],
    mesh=_mesh,
    compiler_params=_cparams,
    scratch_types=[
        pltpu.VMEM((_CHUNK,), jnp.float32),
        pltpu.VMEM((_CHUNK,), jnp.float32),
        pltpu.VMEM((_CHUNK,), jnp.float32),
        pltpu.VMEM((_CHUNK,), jnp.float32),
        pltpu.VMEM((4 * _G,), jnp.float32),
        pltpu.VMEM((_G,), jnp.float32),
        pltpu.VMEM((_CHUNK,), jnp.float32),
        pltpu.VMEM((_CHUNK,), jnp.float32),
        pltpu.VMEM((_CHUNK,), jnp.int32),
        pltpu.VMEM((_RECI,), jnp.int32),
        pltpu.VMEM((_RECF,), jnp.float32),
    ],
)
def _phase1(rx1_hbm, ry1_hbm, rx2_hbm, ry2_hbm, gt_hbm, reci_hbm, recf_hbm,
            x1_v, y1_v, x2_v, y2_v, gt_v, a2_v, bi_v, bu_v, ag_v, ri_v, rf_v):
    wid = lax.axis_index("s") * 2 + lax.axis_index("c")
    base = wid * _CHUNK
    pltpu.sync_copy(rx1_hbm.at[pl.ds(base, _CHUNK)], x1_v)
    pltpu.sync_copy(ry1_hbm.at[pl.ds(base, _CHUNK)], y1_v)
    pltpu.sync_copy(rx2_hbm.at[pl.ds(base, _CHUNK)], x2_v)
    pltpu.sync_copy(ry2_hbm.at[pl.ds(base, _CHUNK)], y2_v)
    pltpu.sync_copy(gt_hbm, gt_v)

    iota = lax.iota(jnp.int32, 16)
    for k in range(_G // 16):
        gx1 = gt_v[pl.ds(k * 16, 16)]
        gy1 = gt_v[pl.ds(_G + k * 16, 16)]
        gx2 = gt_v[pl.ds(2 * _G + k * 16, 16)]
        gy2 = gt_v[pl.ds(3 * _G + k * 16, 16)]
        a2_v[pl.ds(k * 16, 16)] = (gx2 - gx1) * (gy2 - gy1)

    _T = 4

    def vec_body(q, _):
        off = q * (16 * _T)
        x1s = [x1_v[pl.ds(off + t * 16, 16)] for t in range(_T)]
        y1s = [y1_v[pl.ds(off + t * 16, 16)] for t in range(_T)]
        x2s = [x2_v[pl.ds(off + t * 16, 16)] for t in range(_T)]
        y2s = [y2_v[pl.ds(off + t * 16, 16)] for t in range(_T)]
        a1s = [(x2s[t] - x1s[t]) * (y2s[t] - y1s[t]) for t in range(_T)]

        def gt_body(j, carry):
            bis, bus, bas = carry
            js = jnp.full((16,), j, dtype=jnp.int32)
            gx1 = plsc.load_gather(gt_v, [js])
            gy1 = plsc.load_gather(gt_v, [js + _G])
            gx2 = plsc.load_gather(gt_v, [js + 2 * _G])
            gy2 = plsc.load_gather(gt_v, [js + 3 * _G])
            a2 = plsc.load_gather(a2_v, [js])
            nbi, nbu, nba = [], [], []
            for t in range(_T):
                w = jnp.minimum(x2s[t], gx2) - jnp.maximum(x1s[t], gx1)
                h = jnp.minimum(y2s[t], gy2) - jnp.maximum(y1s[t], gy1)
                inter = jnp.maximum(w, 0.0) * jnp.maximum(h, 0.0)
                union = jnp.maximum(a1s[t] + a2 - inter, 1e-8)
                better = inter * bus[t] > bis[t] * union
                nbi.append(jnp.where(better, inter, bis[t]))
                nbu.append(jnp.where(better, union, bus[t]))
                nba.append(jnp.where(better, js, bas[t]))
            return tuple(nbi), tuple(nbu), tuple(nba)

        init = (tuple(jnp.full((16,), -1.0, jnp.float32) for _ in range(_T)),
                tuple(jnp.full((16,), 1.0, jnp.float32) for _ in range(_T)),
                tuple(jnp.zeros((16,), jnp.int32) for _ in range(_T)))
        bis, bus, bas = lax.fori_loop(0, _G, gt_body, init)
        for t in range(_T):
            bi_v[pl.ds(off + t * 16, 16)] = bis[t]
            bu_v[pl.ds(off + t * 16, 16)] = bus[t]
            ag_v[pl.ds(off + t * 16, 16)] = bas[t]
        return 0

    lax.fori_loop(0, _NV // _T, vec_body, 0)

    z16 = jnp.zeros((16,), jnp.int32)
    zf16 = jnp.zeros((16,), jnp.float32)
    for k in range(_RECI // 16):
        ri_v[pl.ds(k * 16, 16)] = z16

    def sel_body(v, carry):
        fgc, bgc = carry
        off = v * 16
        gidx = base + off + iota
        valid = gidx < _N
        bi = bi_v[pl.ds(off, 16)]
        bu = bu_v[pl.ds(off, 16)]
        ag = ag_v[pl.ds(off, 16)]
        x1 = x1_v[pl.ds(off, 16)]
        y1 = y1_v[pl.ds(off, 16)]
        x2 = x2_v[pl.ds(off, 16)]
        y2 = y2_v[pl.ds(off, 16)]
        bi2 = bi + bi
        fgm = (bi2 > bu) & valid
        bgm = (bi2 < bu) & valid
        pf = plsc.cumsum(fgm.astype(jnp.int32)) - 1 + fgc
        mf = fgm & (pf < _MAX_POS)
        plsc.store_scatter(ri_v, [pf + 16], ag, mask=mf)
        plsc.store_scatter(rf_v, [pf], x1, mask=mf)
        plsc.store_scatter(rf_v, [pf + _MAX_POS], y1, mask=mf)
        plsc.store_scatter(rf_v, [pf + 2 * _MAX_POS], x2, mask=mf)
        plsc.store_scatter(rf_v, [pf + 3 * _MAX_POS], y2, mask=mf)
        pb = plsc.cumsum(bgm.astype(jnp.int32)) - 1 + bgc
        mb = bgm & (pb < _N_BG)
        plsc.store_scatter(rf_v, [pb + 128], x1, mask=mb)
        plsc.store_scatter(rf_v, [pb + 128 + _N_BG], y1, mask=mb)
        plsc.store_scatter(rf_v, [pb + 128 + 2 * _N_BG], x2, mask=mb)
        plsc.store_scatter(rf_v, [pb + 128 + 3 * _N_BG], y2, mask=mb)
        fgc = fgc + plsc.all_reduce_population_count(fgm)
        bgc = bgc + plsc.all_reduce_population_count(bgm)
        return fgc, bgc

    fgc, bgc = lax.fori_loop(0, _NV, sel_body, (z16, z16))
    fgc = jnp.minimum(fgc, _MAX_POS)
    bgc = jnp.minimum(bgc, _N_BG)
    arg0 = plsc.load_gather(ag_v, [z16])
    hdr = jnp.where(iota == 0, fgc,
                    jnp.where(iota == 1, bgc,
                              jnp.where(iota == 2, arg0, z16)))
    ri_v[pl.ds(0, 16)] = hdr
    x10 = plsc.load_gather(x1_v, [z16])
    y10 = plsc.load_gather(y1_v, [z16])
    x20 = plsc.load_gather(x2_v, [z16])
    y20 = plsc.load_gather(y2_v, [z16])
    hdrf = jnp.where(iota == 0, x10,
                     jnp.where(iota == 1, y10,
                               jnp.where(iota == 2, x20,
                                         jnp.where(iota == 3, y20, zf16))))
    rf_v[pl.ds(512, 16)] = hdrf
    pltpu.sync_copy(ri_v, reci_hbm.at[pl.ds(wid * _RECI, _RECI)])
    pltpu.sync_copy(rf_v, recf_hbm.at[pl.ds(wid * _RECF, _RECF)])


@functools.partial(
    pl.kernel,
    out_type=[
        jax.ShapeDtypeStruct((_TOTAL, 4), jnp.float32),
        jax.ShapeDtypeStruct((_TOTAL,), jnp.int32),
        jax.ShapeDtypeStruct((_TOTAL * _NUM_CLASSES * 4,), jnp.float32),
        jax.ShapeDtypeStruct((_TOTAL * _NUM_CLASSES * 4,), jnp.float32),
    ],
    mesh=_mesh,
    compiler_params=_cparams,
    scratch_types=[
        pltpu.VMEM((_RECI,), jnp.int32),
        pltpu.VMEM((_RECF,), jnp.float32),
        pltpu.VMEM((_MAX_POS,), jnp.int32),
        pltpu.VMEM((_TOTAL, 4), jnp.float32),
        pltpu.VMEM((4 * _G,), jnp.float32),
        pltpu.VMEM((_G,), jnp.int32),
        pltpu.VMEM((_TOTAL,), jnp.int32),
        pltpu.VMEM((_TOTAL * _NUM_CLASSES * 4,), jnp.float32),
        pltpu.VMEM((_TOTAL * _NUM_CLASSES * 4,), jnp.float32),
    ],
)
def _phase2(reci_hbm, recf_hbm, gt_hbm, glab_hbm,
            orois_hbm, olab_hbm, otgt_hbm, oinw_hbm,
            ci_v, cf_v, fggt_v, rois_v, gtc_v, glab_v, lab_v, tgt_v, inw_v):
    wid = lax.axis_index("s") * 2 + lax.axis_index("c")

    @pl.when(wid == 0)
    def _():
        pltpu.sync_copy(gt_hbm, gtc_v)
        pltpu.sync_copy(glab_hbm, glab_v)
        iota = lax.iota(jnp.int32, 16)
        z16 = jnp.zeros((16,), jnp.int32)

        pltpu.sync_copy(reci_hbm.at[pl.ds(0, _RECI)], ci_v)
        pltpu.sync_copy(recf_hbm.at[pl.ds(0, _RECF)], cf_v)
        arg0 = plsc.load_gather(ci_v, [z16 + 2])
        fggt_v[pl.ds(0, 16)] = arg0
        fggt_v[pl.ds(16, 16)] = arg0
        for c in range(4):
            r0c = plsc.load_gather(cf_v, [z16 + 512 + c])
            for s in range(_TOTAL // 16):
                plsc.store_scatter(rois_v, [iota + s * 16, z16 + c], r0c)

        def merge_cond(carry):
            ci, fo, bo = carry
            return (ci < _NW) & ((fo < _MAX_POS) | (bo < _N_BG))

        def merge_body(carry):
            ci, fo, bo = carry
            pltpu.sync_copy(reci_hbm.at[pl.ds(ci * _RECI, _RECI)], ci_v)
            pltpu.sync_copy(recf_hbm.at[pl.ds(ci * _RECF, _RECF)], cf_v)
            hv = ci_v[pl.ds(0, 16)]
            cfg = hv[0]
            cbg = hv[1]
            cfgv = jnp.full((16,), cfg, dtype=jnp.int32)
            cbgv = jnp.full((16,), cbg, dtype=jnp.int32)
            fov = jnp.full((16,), fo, dtype=jnp.int32)
            bov = jnp.full((16,), bo, dtype=jnp.int32)
            for k in range(_MAX_POS // 16):
                lpos = iota + k * 16
                dpos = fov + lpos
                m = (lpos < cfgv) & (dpos < _MAX_POS)
                gts = plsc.load_gather(ci_v, [16 + lpos])
                plsc.store_scatter(fggt_v, [dpos], gts, mask=m)
                for c in range(4):
                    cv = plsc.load_gather(cf_v, [lpos + c * _MAX_POS])
                    plsc.store_scatter(rois_v, [dpos, z16 + c], cv, mask=m)
            for k in range(_N_BG // 16):
                lpos = iota + k * 16
                dpos = bov + lpos
                m = (lpos < cbgv) & (dpos < _N_BG)
                for c in range(4):
                    cv = plsc.load_gather(cf_v, [lpos + 128 + c * _N_BG])
                    plsc.store_scatter(rois_v, [dpos + _MAX_POS, z16 + c],
                                       cv, mask=m)
            return ci + 1, fo + cfg, bo + cbg

        lax.while_loop(merge_cond, merge_body, (0, 0, 0))
        pltpu.sync_copy(rois_v, orois_hbm)

        zf = jnp.zeros((16,), jnp.float32)

        def zero_body(i, _):
            for u in range(4):
                tgt_v[pl.ds((i * 4 + u) * 16, 16)] = zf
                inw_v[pl.ds((i * 4 + u) * 16, 16)] = zf
            return 0

        lax.fori_loop(0, _TOTAL * _NUM_CLASSES * 4 // 64, zero_body, 0)

        one = jnp.ones((16,), jnp.float32)
        for k in range(_MAX_POS // 16):
            rows = iota + k * 16
            fgt = fggt_v[pl.ds(k * 16, 16)]
            lab = plsc.load_gather(glab_v, [fgt])
            lab_v[pl.ds(k * 16, 16)] = lab
            px1 = plsc.load_gather(rois_v, [rows, z16])
            py1 = plsc.load_gather(rois_v, [rows, z16 + 1])
            px2 = plsc.load_gather(rois_v, [rows, z16 + 2])
            py2 = plsc.load_gather(rois_v, [rows, z16 + 3])
            gx1 = plsc.load_gather(gtc_v, [fgt])
            gy1 = plsc.load_gather(gtc_v, [fgt + _G])
            gx2 = plsc.load_gather(gtc_v, [fgt + 2 * _G])
            gy2 = plsc.load_gather(gtc_v, [fgt + 3 * _G])
            pw = px2 - px1 + 1.0
            ph = py2 - py1 + 1.0
            pxc = px1 + 0.5 * pw
            pyc = py1 + 0.5 * ph
            gw = gx2 - gx1 + 1.0
            gh = gy2 - gy1 + 1.0
            gxc = gx1 + 0.5 * gw
            gyc = gy1 + 0.5 * gh
            dx = (gxc - pxc) / pw
            dy = (gyc - pyc) / ph
            dw = _vlog(gw / pw)
            dh = _vlog(gh / ph)
            flat = rows * (_NUM_CLASSES * 4) + lab * 4
            plsc.store_scatter(tgt_v, [flat], dx)
            plsc.store_scatter(tgt_v, [flat + 1], dy)
            plsc.store_scatter(tgt_v, [flat + 2], dw)
            plsc.store_scatter(tgt_v, [flat + 3], dh)
            plsc.store_scatter(inw_v, [flat], one)
            plsc.store_scatter(inw_v, [flat + 1], one)
            plsc.store_scatter(inw_v, [flat + 2], one)
            plsc.store_scatter(inw_v, [flat + 3], one)

        for k in range(2, _TOTAL // 16):
            lab_v[pl.ds(k * 16, 16)] = z16
        pltpu.sync_copy(lab_v, olab_hbm)
        pltpu.sync_copy(tgt_v, otgt_hbm)
        pltpu.sync_copy(inw_v, oinw_hbm)


def kernel(rois, gt_bboxes, gt_labels):
    rois_pad = jnp.pad(rois, ((0, _NPAD - _N), (0, 0)))
    gt_cols = gt_bboxes.T.reshape(-1)
    reci, recf = _phase1(rois_pad[:, 0], rois_pad[:, 1],
                         rois_pad[:, 2], rois_pad[:, 3], gt_cols)
    frois, flab, ftgt, finw = _phase2(
        reci, recf, gt_cols, gt_labels.astype(jnp.int32))
    ftgt = ftgt.reshape(_TOTAL, _NUM_CLASSES * 4)
    finw = finw.reshape(_TOTAL, _NUM_CLASSES * 4)
    foutw = jnp.ones((_TOTAL, _NUM_CLASSES * 4), jnp.float32)
    return frois, flab, ftgt, finw, foutw


# speedup vs baseline: 1.3999x; 1.0027x over previous
import functools

import jax
import jax.numpy as jnp
from jax import lax
from jax.experimental import pallas as pl
from jax.experimental.pallas import tpu as pltpu
from jax.experimental.pallas import tpu_sc as plsc

_NUM_CLASSES = 21
_N = 20000
_NPAD = 20480
_CHUNK = 640
_NV = _CHUNK // 16
_G = 64
_MAX_POS = 32
_N_BG = 96
_TOTAL = 128
_RECI = 64
_RECF = 528
_NW = 32

_mesh = plsc.VectorSubcoreMesh(core_axis_name="c", subcore_axis_name="s",
                               num_cores=2)
_cparams = pltpu.CompilerParams(needs_layout_passes=False)


def _vlog(x):
    bits = plsc.bitcast(x, jnp.int32)
    e = ((bits >> 23) & 0xFF) - 127
    m = plsc.bitcast((bits & 0x7FFFFF) | 0x3F800000, jnp.float32)
    big = m > 1.4142135623730951
    m = jnp.where(big, m * 0.5, m)
    e = jnp.where(big, e + 1, e)
    r = (m - 1.0) / (m + 1.0)
    r2 = r * r
    p = 1.0 / 7.0 + r2 * (1.0 / 9.0)
    p = 1.0 / 5.0 + r2 * p
    p = 1.0 / 3.0 + r2 * p
    p = 1.0 + r2 * p
    return e.astype(jnp.float32) * 0.6931471805599453 + 2.0 * r * p


@functools.partial(
    pl.kernel,
    out_type=[
        jax.ShapeDtypeStruct((_NW * _RECI,), jnp.int32),
        jax.ShapeDtypeStruct((_NW * _RECF,), jnp.float32),
    ],
    mesh=_mesh,
    compiler_params=_cparams,
    scratch_types=[
        pltpu.VMEM((_CHUNK,), jnp.float32),
        pltpu.VMEM((_CHUNK,), jnp.float32),
        pltpu.VMEM((_CHUNK,), jnp.float32),
        pltpu.VMEM((_CHUNK,), jnp.float32),
        pltpu.VMEM((4 * _G,), jnp.float32),
        pltpu.VMEM((_G,), jnp.float32),
        pltpu.VMEM((_CHUNK,), jnp.float32),
        pltpu.VMEM((_CHUNK,), jnp.float32),
        pltpu.VMEM((_CHUNK,), jnp.int32),
        pltpu.VMEM((_RECI,), jnp.int32),
        pltpu.VMEM((_RECF,), jnp.float32),
    ],
)
def _phase1(rx1_hbm, ry1_hbm, rx2_hbm, ry2_hbm, gt_hbm, reci_hbm, recf_hbm,
            x1_v, y1_v, x2_v, y2_v, gt_v, a2_v, bi_v, bu_v, ag_v, ri_v, rf_v):
    wid = lax.axis_index("s") * 2 + lax.axis_index("c")
    base = wid * _CHUNK
    pltpu.sync_copy(rx1_hbm.at[pl.ds(base, _CHUNK)], x1_v)
    pltpu.sync_copy(ry1_hbm.at[pl.ds(base, _CHUNK)], y1_v)
    pltpu.sync_copy(rx2_hbm.at[pl.ds(base, _CHUNK)], x2_v)
    pltpu.sync_copy(ry2_hbm.at[pl.ds(base, _CHUNK)], y2_v)
    pltpu.sync_copy(gt_hbm, gt_v)

    iota = lax.iota(jnp.int32, 16)
    for k in range(_G // 16):
        gx1 = gt_v[pl.ds(k * 16, 16)]
        gy1 = gt_v[pl.ds(_G + k * 16, 16)]
        gx2 = gt_v[pl.ds(2 * _G + k * 16, 16)]
        gy2 = gt_v[pl.ds(3 * _G + k * 16, 16)]
        a2_v[pl.ds(k * 16, 16)] = (gx2 - gx1) * (gy2 - gy1)

    _T = 4

    def vec_body(q, _):
        off = q * (16 * _T)
        x1s = [x1_v[pl.ds(off + t * 16, 16)] for t in range(_T)]
        y1s = [y1_v[pl.ds(off + t * 16, 16)] for t in range(_T)]
        x2s = [x2_v[pl.ds(off + t * 16, 16)] for t in range(_T)]
        y2s = [y2_v[pl.ds(off + t * 16, 16)] for t in range(_T)]
        a1s = [(x2s[t] - x1s[t]) * (y2s[t] - y1s[t]) for t in range(_T)]

        def gt_body(j2, carry):
            bis, bus, bas = list(carry[0]), list(carry[1]), list(carry[2])
            for u in range(2):
                j = j2 * 2 + u
                js = jnp.full((16,), j, dtype=jnp.int32)
                gx1 = plsc.load_gather(gt_v, [js])
                gy1 = plsc.load_gather(gt_v, [js + _G])
                gx2 = plsc.load_gather(gt_v, [js + 2 * _G])
                gy2 = plsc.load_gather(gt_v, [js + 3 * _G])
                a2 = plsc.load_gather(a2_v, [js])
                for t in range(_T):
                    w = jnp.minimum(x2s[t], gx2) - jnp.maximum(x1s[t], gx1)
                    h = jnp.minimum(y2s[t], gy2) - jnp.maximum(y1s[t], gy1)
                    inter = jnp.maximum(w, 0.0) * jnp.maximum(h, 0.0)
                    union = jnp.maximum(a1s[t] + a2 - inter, 1e-8)
                    better = inter * bus[t] > bis[t] * union
                    bis[t] = jnp.where(better, inter, bis[t])
                    bus[t] = jnp.where(better, union, bus[t])
                    bas[t] = jnp.where(better, js, bas[t])
            return tuple(bis), tuple(bus), tuple(bas)

        init = (tuple(jnp.full((16,), -1.0, jnp.float32) for _ in range(_T)),
                tuple(jnp.full((16,), 1.0, jnp.float32) for _ in range(_T)),
                tuple(jnp.zeros((16,), jnp.int32) for _ in range(_T)))
        bis, bus, bas = lax.fori_loop(0, _G // 2, gt_body, init)
        for t in range(_T):
            bi_v[pl.ds(off + t * 16, 16)] = bis[t]
            bu_v[pl.ds(off + t * 16, 16)] = bus[t]
            ag_v[pl.ds(off + t * 16, 16)] = bas[t]
        return 0

    lax.fori_loop(0, _NV // _T, vec_body, 0)

    z16 = jnp.zeros((16,), jnp.int32)
    zf16 = jnp.zeros((16,), jnp.float32)
    for k in range(_RECI // 16):
        ri_v[pl.ds(k * 16, 16)] = z16

    def sel_body(v, carry):
        fgc, bgc = carry
        off = v * 16
        gidx = base + off + iota
        valid = gidx < _N
        bi = bi_v[pl.ds(off, 16)]
        bu = bu_v[pl.ds(off, 16)]
        ag = ag_v[pl.ds(off, 16)]
        x1 = x1_v[pl.ds(off, 16)]
        y1 = y1_v[pl.ds(off, 16)]
        x2 = x2_v[pl.ds(off, 16)]
        y2 = y2_v[pl.ds(off, 16)]
        bi2 = bi + bi
        fgm = (bi2 > bu) & valid
        bgm = (bi2 < bu) & valid
        pf = plsc.cumsum(fgm.astype(jnp.int32)) - 1 + fgc
        mf = fgm & (pf < _MAX_POS)
        plsc.store_scatter(ri_v, [pf + 16], ag, mask=mf)
        plsc.store_scatter(rf_v, [pf], x1, mask=mf)
        plsc.store_scatter(rf_v, [pf + _MAX_POS], y1, mask=mf)
        plsc.store_scatter(rf_v, [pf + 2 * _MAX_POS], x2, mask=mf)
        plsc.store_scatter(rf_v, [pf + 3 * _MAX_POS], y2, mask=mf)
        pb = plsc.cumsum(bgm.astype(jnp.int32)) - 1 + bgc
        mb = bgm & (pb < _N_BG)
        plsc.store_scatter(rf_v, [pb + 128], x1, mask=mb)
        plsc.store_scatter(rf_v, [pb + 128 + _N_BG], y1, mask=mb)
        plsc.store_scatter(rf_v, [pb + 128 + 2 * _N_BG], x2, mask=mb)
        plsc.store_scatter(rf_v, [pb + 128 + 3 * _N_BG], y2, mask=mb)
        fgc = fgc + plsc.all_reduce_population_count(fgm)
        bgc = bgc + plsc.all_reduce_population_count(bgm)
        return fgc, bgc

    fgc, bgc = lax.fori_loop(0, _NV, sel_body, (z16, z16))
    fgc = jnp.minimum(fgc, _MAX_POS)
    bgc = jnp.minimum(bgc, _N_BG)
    arg0 = plsc.load_gather(ag_v, [z16])
    hdr = jnp.where(iota == 0, fgc,
                    jnp.where(iota == 1, bgc,
                              jnp.where(iota == 2, arg0, z16)))
    ri_v[pl.ds(0, 16)] = hdr
    x10 = plsc.load_gather(x1_v, [z16])
    y10 = plsc.load_gather(y1_v, [z16])
    x20 = plsc.load_gather(x2_v, [z16])
    y20 = plsc.load_gather(y2_v, [z16])
    hdrf = jnp.where(iota == 0, x10,
                     jnp.where(iota == 1, y10,
                               jnp.where(iota == 2, x20,
                                         jnp.where(iota == 3, y20, zf16))))
    rf_v[pl.ds(512, 16)] = hdrf
    pltpu.sync_copy(ri_v, reci_hbm.at[pl.ds(wid * _RECI, _RECI)])
    pltpu.sync_copy(rf_v, recf_hbm.at[pl.ds(wid * _RECF, _RECF)])


@functools.partial(
    pl.kernel,
    out_type=[
        jax.ShapeDtypeStruct((_TOTAL, 4), jnp.float32),
        jax.ShapeDtypeStruct((_TOTAL,), jnp.int32),
        jax.ShapeDtypeStruct((_TOTAL * _NUM_CLASSES * 4,), jnp.float32),
        jax.ShapeDtypeStruct((_TOTAL * _NUM_CLASSES * 4,), jnp.float32),
    ],
    mesh=_mesh,
    compiler_params=_cparams,
    scratch_types=[
        pltpu.VMEM((_RECI,), jnp.int32),
        pltpu.VMEM((_RECF,), jnp.float32),
        pltpu.VMEM((_MAX_POS,), jnp.int32),
        pltpu.VMEM((_TOTAL, 4), jnp.float32),
        pltpu.VMEM((4 * _G,), jnp.float32),
        pltpu.VMEM((_G,), jnp.int32),
        pltpu.VMEM((_TOTAL,), jnp.int32),
        pltpu.VMEM((_TOTAL * _NUM_CLASSES * 4,), jnp.float32),
        pltpu.VMEM((_TOTAL * _NUM_CLASSES * 4,), jnp.float32),
    ],
)
def _phase2(reci_hbm, recf_hbm, gt_hbm, glab_hbm,
            orois_hbm, olab_hbm, otgt_hbm, oinw_hbm,
            ci_v, cf_v, fggt_v, rois_v, gtc_v, glab_v, lab_v, tgt_v, inw_v):
    wid = lax.axis_index("s") * 2 + lax.axis_index("c")

    @pl.when(wid == 0)
    def _():
        pltpu.sync_copy(gt_hbm, gtc_v)
        pltpu.sync_copy(glab_hbm, glab_v)
        iota = lax.iota(jnp.int32, 16)
        z16 = jnp.zeros((16,), jnp.int32)

        pltpu.sync_copy(reci_hbm.at[pl.ds(0, _RECI)], ci_v)
        pltpu.sync_copy(recf_hbm.at[pl.ds(0, _RECF)], cf_v)
        arg0 = plsc.load_gather(ci_v, [z16 + 2])
        fggt_v[pl.ds(0, 16)] = arg0
        fggt_v[pl.ds(16, 16)] = arg0
        for c in range(4):
            r0c = plsc.load_gather(cf_v, [z16 + 512 + c])
            for s in range(_TOTAL // 16):
                plsc.store_scatter(rois_v, [iota + s * 16, z16 + c], r0c)

        def merge_cond(carry):
            ci, fo, bo = carry
            return (ci < _NW) & ((fo < _MAX_POS) | (bo < _N_BG))

        def merge_body(carry):
            ci, fo, bo = carry
            pltpu.sync_copy(reci_hbm.at[pl.ds(ci * _RECI, _RECI)], ci_v)
            pltpu.sync_copy(recf_hbm.at[pl.ds(ci * _RECF, _RECF)], cf_v)
            hv = ci_v[pl.ds(0, 16)]
            cfg = hv[0]
            cbg = hv[1]
            cfgv = jnp.full((16,), cfg, dtype=jnp.int32)
            cbgv = jnp.full((16,), cbg, dtype=jnp.int32)
            fov = jnp.full((16,), fo, dtype=jnp.int32)
            bov = jnp.full((16,), bo, dtype=jnp.int32)
            for k in range(_MAX_POS // 16):
                lpos = iota + k * 16
                dpos = fov + lpos
                m = (lpos < cfgv) & (dpos < _MAX_POS)
                gts = plsc.load_gather(ci_v, [16 + lpos])
                plsc.store_scatter(fggt_v, [dpos], gts, mask=m)
                for c in range(4):
                    cv = plsc.load_gather(cf_v, [lpos + c * _MAX_POS])
                    plsc.store_scatter(rois_v, [dpos, z16 + c], cv, mask=m)
            for k in range(_N_BG // 16):
                lpos = iota + k * 16
                dpos = bov + lpos
                m = (lpos < cbgv) & (dpos < _N_BG)
                for c in range(4):
                    cv = plsc.load_gather(cf_v, [lpos + 128 + c * _N_BG])
                    plsc.store_scatter(rois_v, [dpos + _MAX_POS, z16 + c],
                                       cv, mask=m)
            return ci + 1, fo + cfg, bo + cbg

        lax.while_loop(merge_cond, merge_body, (0, 0, 0))
        pltpu.sync_copy(rois_v, orois_hbm)

        zf = jnp.zeros((16,), jnp.float32)

        def zero_body(i, _):
            for u in range(4):
                tgt_v[pl.ds((i * 4 + u) * 16, 16)] = zf
                inw_v[pl.ds((i * 4 + u) * 16, 16)] = zf
            return 0

        lax.fori_loop(0, _TOTAL * _NUM_CLASSES * 4 // 64, zero_body, 0)

        one = jnp.ones((16,), jnp.float32)
        for k in range(_MAX_POS // 16):
            rows = iota + k * 16
            fgt = fggt_v[pl.ds(k * 16, 16)]
            lab = plsc.load_gather(glab_v, [fgt])
            lab_v[pl.ds(k * 16, 16)] = lab
            px1 = plsc.load_gather(rois_v, [rows, z16])
            py1 = plsc.load_gather(rois_v, [rows, z16 + 1])
            px2 = plsc.load_gather(rois_v, [rows, z16 + 2])
            py2 = plsc.load_gather(rois_v, [rows, z16 + 3])
            gx1 = plsc.load_gather(gtc_v, [fgt])
            gy1 = plsc.load_gather(gtc_v, [fgt + _G])
            gx2 = plsc.load_gather(gtc_v, [fgt + 2 * _G])
            gy2 = plsc.load_gather(gtc_v, [fgt + 3 * _G])
            pw = px2 - px1 + 1.0
            ph = py2 - py1 + 1.0
            pxc = px1 + 0.5 * pw
            pyc = py1 + 0.5 * ph
            gw = gx2 - gx1 + 1.0
            gh = gy2 - gy1 + 1.0
            gxc = gx1 + 0.5 * gw
            gyc = gy1 + 0.5 * gh
            dx = (gxc - pxc) / pw
            dy = (gyc - pyc) / ph
            dw = _vlog(gw / pw)
            dh = _vlog(gh / ph)
            flat = rows * (_NUM_CLASSES * 4) + lab * 4
            plsc.store_scatter(tgt_v, [flat], dx)
            plsc.store_scatter(tgt_v, [flat + 1], dy)
            plsc.store_scatter(tgt_v, [flat + 2], dw)
            plsc.store_scatter(tgt_v, [flat + 3], dh)
            plsc.store_scatter(inw_v, [flat], one)
            plsc.store_scatter(inw_v, [flat + 1], one)
            plsc.store_scatter(inw_v, [flat + 2], one)
            plsc.store_scatter(inw_v, [flat + 3], one)

        for k in range(2, _TOTAL // 16):
            lab_v[pl.ds(k * 16, 16)] = z16
        pltpu.sync_copy(lab_v, olab_hbm)
        pltpu.sync_copy(tgt_v, otgt_hbm)
        pltpu.sync_copy(inw_v, oinw_hbm)


def kernel(rois, gt_bboxes, gt_labels):
    rois_pad = jnp.pad(rois, ((0, _NPAD - _N), (0, 0)))
    gt_cols = gt_bboxes.T.reshape(-1)
    reci, recf = _phase1(rois_pad[:, 0], rois_pad[:, 1],
                         rois_pad[:, 2], rois_pad[:, 3], gt_cols)
    frois, flab, ftgt, finw = _phase2(
        reci, recf, gt_cols, gt_labels.astype(jnp.int32))
    ftgt = ftgt.reshape(_TOTAL, _NUM_CLASSES * 4)
    finw = finw.reshape(_TOTAL, _NUM_CLASSES * 4)
    foutw = jnp.ones((_TOTAL, _NUM_CLASSES * 4), jnp.float32)
    return frois, flab, ftgt, finw, foutw


# speedup vs baseline: 1.4000x; 1.0001x over previous
import functools

import jax
import jax.numpy as jnp
from jax import lax
from jax.experimental import pallas as pl
from jax.experimental.pallas import tpu as pltpu
from jax.experimental.pallas import tpu_sc as plsc

_NUM_CLASSES = 21
_N = 20000
_NPAD = 20480
_CHUNK = 640
_NV = _CHUNK // 16
_G = 64
_MAX_POS = 32
_N_BG = 96
_TOTAL = 128
_RECI = 64
_RECF = 528
_NW = 32

_mesh = plsc.VectorSubcoreMesh(core_axis_name="c", subcore_axis_name="s",
                               num_cores=2)
_cparams = pltpu.CompilerParams(needs_layout_passes=False)


def _vlog(x):
    bits = plsc.bitcast(x, jnp.int32)
    e = ((bits >> 23) & 0xFF) - 127
    m = plsc.bitcast((bits & 0x7FFFFF) | 0x3F800000, jnp.float32)
    big = m > 1.4142135623730951
    m = jnp.where(big, m * 0.5, m)
    e = jnp.where(big, e + 1, e)
    r = (m - 1.0) / (m + 1.0)
    r2 = r * r
    p = 1.0 / 7.0 + r2 * (1.0 / 9.0)
    p = 1.0 / 5.0 + r2 * p
    p = 1.0 / 3.0 + r2 * p
    p = 1.0 + r2 * p
    return e.astype(jnp.float32) * 0.6931471805599453 + 2.0 * r * p


@functools.partial(
    pl.kernel,
    out_type=[
        jax.ShapeDtypeStruct((_NW * _RECI,), jnp.int32),
        jax.ShapeDtypeStruct((_NW * _RECF,), jnp.float32),
    ],
    mesh=_mesh,
    compiler_params=_cparams,
    scratch_types=[
        pltpu.VMEM((_CHUNK,), jnp.float32),
        pltpu.VMEM((_CHUNK,), jnp.float32),
        pltpu.VMEM((_CHUNK,), jnp.float32),
        pltpu.VMEM((_CHUNK,), jnp.float32),
        pltpu.VMEM((4 * _G,), jnp.float32),
        pltpu.VMEM((_G,), jnp.float32),
        pltpu.VMEM((_CHUNK,), jnp.float32),
        pltpu.VMEM((_CHUNK,), jnp.float32),
        pltpu.VMEM((_CHUNK,), jnp.int32),
        pltpu.VMEM((_RECI,), jnp.int32),
        pltpu.VMEM((_RECF,), jnp.float32),
    ],
)
def _phase1(rx1_hbm, ry1_hbm, rx2_hbm, ry2_hbm, gt_hbm, reci_hbm, recf_hbm,
            x1_v, y1_v, x2_v, y2_v, gt_v, a2_v, bi_v, bu_v, ag_v, ri_v, rf_v):
    wid = lax.axis_index("s") * 2 + lax.axis_index("c")
    base = wid * _CHUNK
    pltpu.sync_copy(rx1_hbm.at[pl.ds(base, _CHUNK)], x1_v)
    pltpu.sync_copy(ry1_hbm.at[pl.ds(base, _CHUNK)], y1_v)
    pltpu.sync_copy(rx2_hbm.at[pl.ds(base, _CHUNK)], x2_v)
    pltpu.sync_copy(ry2_hbm.at[pl.ds(base, _CHUNK)], y2_v)
    pltpu.sync_copy(gt_hbm, gt_v)

    iota = lax.iota(jnp.int32, 16)
    for k in range(_G // 16):
        gx1 = gt_v[pl.ds(k * 16, 16)]
        gy1 = gt_v[pl.ds(_G + k * 16, 16)]
        gx2 = gt_v[pl.ds(2 * _G + k * 16, 16)]
        gy2 = gt_v[pl.ds(3 * _G + k * 16, 16)]
        a2_v[pl.ds(k * 16, 16)] = (gx2 - gx1) * (gy2 - gy1)

    _T = 4

    def vec_body(q, _):
        off = q * (16 * _T)
        x1s = [x1_v[pl.ds(off + t * 16, 16)] for t in range(_T)]
        y1s = [y1_v[pl.ds(off + t * 16, 16)] for t in range(_T)]
        x2s = [x2_v[pl.ds(off + t * 16, 16)] for t in range(_T)]
        y2s = [y2_v[pl.ds(off + t * 16, 16)] for t in range(_T)]
        a1s = [(x2s[t] - x1s[t]) * (y2s[t] - y1s[t]) for t in range(_T)]

        def gt_body(j, carry):
            bis, bus, bas = carry
            js = jnp.full((16,), j, dtype=jnp.int32)
            gx1 = plsc.load_gather(gt_v, [js])
            gy1 = plsc.load_gather(gt_v, [js + _G])
            gx2 = plsc.load_gather(gt_v, [js + 2 * _G])
            gy2 = plsc.load_gather(gt_v, [js + 3 * _G])
            a2 = plsc.load_gather(a2_v, [js])
            nbi, nbu, nba = [], [], []
            for t in range(_T):
                w = jnp.minimum(x2s[t], gx2) - jnp.maximum(x1s[t], gx1)
                h = jnp.minimum(y2s[t], gy2) - jnp.maximum(y1s[t], gy1)
                inter = jnp.maximum(w, 0.0) * jnp.maximum(h, 0.0)
                union = jnp.maximum(a1s[t] + a2 - inter, 1e-8)
                better = inter * bus[t] > bis[t] * union
                nbi.append(jnp.where(better, inter, bis[t]))
                nbu.append(jnp.where(better, union, bus[t]))
                nba.append(jnp.where(better, js, bas[t]))
            return tuple(nbi), tuple(nbu), tuple(nba)

        init = (tuple(jnp.full((16,), -1.0, jnp.float32) for _ in range(_T)),
                tuple(jnp.full((16,), 1.0, jnp.float32) for _ in range(_T)),
                tuple(jnp.zeros((16,), jnp.int32) for _ in range(_T)))
        bis, bus, bas = lax.fori_loop(0, _G, gt_body, init)
        for t in range(_T):
            bi_v[pl.ds(off + t * 16, 16)] = bis[t]
            bu_v[pl.ds(off + t * 16, 16)] = bus[t]
            ag_v[pl.ds(off + t * 16, 16)] = bas[t]
        return 0

    lax.fori_loop(0, _NV // _T, vec_body, 0)

    z16 = jnp.zeros((16,), jnp.int32)
    zf16 = jnp.zeros((16,), jnp.float32)
    for k in range(_RECI // 16):
        ri_v[pl.ds(k * 16, 16)] = z16

    def sel_body(v, carry):
        fgc, bgc = carry
        off = v * 16
        gidx = base + off + iota
        valid = gidx < _N
        bi = bi_v[pl.ds(off, 16)]
        bu = bu_v[pl.ds(off, 16)]
        ag = ag_v[pl.ds(off, 16)]
        x1 = x1_v[pl.ds(off, 16)]
        y1 = y1_v[pl.ds(off, 16)]
        x2 = x2_v[pl.ds(off, 16)]
        y2 = y2_v[pl.ds(off, 16)]
        bi2 = bi + bi
        fgm = (bi2 > bu) & valid
        bgm = (bi2 < bu) & valid
        pf = plsc.cumsum(fgm.astype(jnp.int32)) - 1 + fgc
        mf = fgm & (pf < _MAX_POS)
        plsc.store_scatter(ri_v, [pf + 16], ag, mask=mf)
        plsc.store_scatter(rf_v, [pf], x1, mask=mf)
        plsc.store_scatter(rf_v, [pf + _MAX_POS], y1, mask=mf)
        plsc.store_scatter(rf_v, [pf + 2 * _MAX_POS], x2, mask=mf)
        plsc.store_scatter(rf_v, [pf + 3 * _MAX_POS], y2, mask=mf)
        pb = plsc.cumsum(bgm.astype(jnp.int32)) - 1 + bgc
        mb = bgm & (pb < _N_BG)
        plsc.store_scatter(rf_v, [pb + 128], x1, mask=mb)
        plsc.store_scatter(rf_v, [pb + 128 + _N_BG], y1, mask=mb)
        plsc.store_scatter(rf_v, [pb + 128 + 2 * _N_BG], x2, mask=mb)
        plsc.store_scatter(rf_v, [pb + 128 + 3 * _N_BG], y2, mask=mb)
        fgc = fgc + plsc.all_reduce_population_count(fgm)
        bgc = bgc + plsc.all_reduce_population_count(bgm)
        return fgc, bgc

    fgc, bgc = lax.fori_loop(0, _NV, sel_body, (z16, z16))
    fgc = jnp.minimum(fgc, _MAX_POS)
    bgc = jnp.minimum(bgc, _N_BG)
    arg0 = plsc.load_gather(ag_v, [z16])
    hdr = jnp.where(iota == 0, fgc,
                    jnp.where(iota == 1, bgc,
                              jnp.where(iota == 2, arg0, z16)))
    ri_v[pl.ds(0, 16)] = hdr
    x10 = plsc.load_gather(x1_v, [z16])
    y10 = plsc.load_gather(y1_v, [z16])
    x20 = plsc.load_gather(x2_v, [z16])
    y20 = plsc.load_gather(y2_v, [z16])
    hdrf = jnp.where(iota == 0, x10,
                     jnp.where(iota == 1, y10,
                               jnp.where(iota == 2, x20,
                                         jnp.where(iota == 3, y20, zf16))))
    rf_v[pl.ds(512, 16)] = hdrf
    pltpu.sync_copy(ri_v, reci_hbm.at[pl.ds(wid * _RECI, _RECI)])
    pltpu.sync_copy(rf_v, recf_hbm.at[pl.ds(wid * _RECF, _RECF)])


@functools.partial(
    pl.kernel,
    out_type=[
        jax.ShapeDtypeStruct((_TOTAL, 4), jnp.float32),
        jax.ShapeDtypeStruct((_TOTAL,), jnp.int32),
        jax.ShapeDtypeStruct((_TOTAL * _NUM_CLASSES * 4,), jnp.float32),
        jax.ShapeDtypeStruct((_TOTAL * _NUM_CLASSES * 4,), jnp.float32),
    ],
    mesh=_mesh,
    compiler_params=_cparams,
    scratch_types=[
        pltpu.VMEM((_RECI,), jnp.int32),
        pltpu.VMEM((_RECF,), jnp.float32),
        pltpu.VMEM((_MAX_POS,), jnp.int32),
        pltpu.VMEM((_TOTAL, 4), jnp.float32),
        pltpu.VMEM((4 * _G,), jnp.float32),
        pltpu.VMEM((_G,), jnp.int32),
        pltpu.VMEM((_TOTAL,), jnp.int32),
        pltpu.VMEM((_TOTAL * _NUM_CLASSES * 4,), jnp.float32),
        pltpu.VMEM((_TOTAL * _NUM_CLASSES * 4,), jnp.float32),
    ],
)
def _phase2(reci_hbm, recf_hbm, gt_hbm, glab_hbm,
            orois_hbm, olab_hbm, otgt_hbm, oinw_hbm,
            ci_v, cf_v, fggt_v, rois_v, gtc_v, glab_v, lab_v, tgt_v, inw_v):
    wid = lax.axis_index("s") * 2 + lax.axis_index("c")

    @pl.when(wid == 0)
    def _():
        pltpu.sync_copy(gt_hbm, gtc_v)
        pltpu.sync_copy(glab_hbm, glab_v)
        iota = lax.iota(jnp.int32, 16)
        z16 = jnp.zeros((16,), jnp.int32)

        pltpu.sync_copy(reci_hbm.at[pl.ds(0, _RECI)], ci_v)
        pltpu.sync_copy(recf_hbm.at[pl.ds(0, _RECF)], cf_v)
        arg0 = plsc.load_gather(ci_v, [z16 + 2])
        fggt_v[pl.ds(0, 16)] = arg0
        fggt_v[pl.ds(16, 16)] = arg0
        for c in range(4):
            r0c = plsc.load_gather(cf_v, [z16 + 512 + c])
            for s in range(_TOTAL // 16):
                plsc.store_scatter(rois_v, [iota + s * 16, z16 + c], r0c)

        def merge_cond(carry):
            ci, fo, bo = carry
            return (ci < _NW) & ((fo < _MAX_POS) | (bo < _N_BG))

        def merge_body(carry):
            ci, fo, bo = carry
            pltpu.sync_copy(reci_hbm.at[pl.ds(ci * _RECI, _RECI)], ci_v)
            pltpu.sync_copy(recf_hbm.at[pl.ds(ci * _RECF, _RECF)], cf_v)
            hv = ci_v[pl.ds(0, 16)]
            cfg = hv[0]
            cbg = hv[1]
            cfgv = jnp.full((16,), cfg, dtype=jnp.int32)
            cbgv = jnp.full((16,), cbg, dtype=jnp.int32)
            fov = jnp.full((16,), fo, dtype=jnp.int32)
            bov = jnp.full((16,), bo, dtype=jnp.int32)
            for k in range(_MAX_POS // 16):
                lpos = iota + k * 16
                dpos = fov + lpos
                m = (lpos < cfgv) & (dpos < _MAX_POS)
                gts = plsc.load_gather(ci_v, [16 + lpos])
                plsc.store_scatter(fggt_v, [dpos], gts, mask=m)
                for c in range(4):
                    cv = plsc.load_gather(cf_v, [lpos + c * _MAX_POS])
                    plsc.store_scatter(rois_v, [dpos, z16 + c], cv, mask=m)
            for k in range(_N_BG // 16):
                lpos = iota + k * 16
                dpos = bov + lpos
                m = (lpos < cbgv) & (dpos < _N_BG)
                for c in range(4):
                    cv = plsc.load_gather(cf_v, [lpos + 128 + c * _N_BG])
                    plsc.store_scatter(rois_v, [dpos + _MAX_POS, z16 + c],
                                       cv, mask=m)
            return ci + 1, fo + cfg, bo + cbg

        lax.while_loop(merge_cond, merge_body, (0, 0, 0))
        pltpu.sync_copy(rois_v, orois_hbm)

        zf = jnp.zeros((16,), jnp.float32)

        def zero_body(i, _):
            for u in range(4):
                tgt_v[pl.ds((i * 4 + u) * 16, 16)] = zf
                inw_v[pl.ds((i * 4 + u) * 16, 16)] = zf
            return 0

        lax.fori_loop(0, _TOTAL * _NUM_CLASSES * 4 // 64, zero_body, 0)

        one = jnp.ones((16,), jnp.float32)
        for k in range(_MAX_POS // 16):
            rows = iota + k * 16
            fgt = fggt_v[pl.ds(k * 16, 16)]
            lab = plsc.load_gather(glab_v, [fgt])
            lab_v[pl.ds(k * 16, 16)] = lab
            px1 = plsc.load_gather(rois_v, [rows, z16])
            py1 = plsc.load_gather(rois_v, [rows, z16 + 1])
            px2 = plsc.load_gather(rois_v, [rows, z16 + 2])
            py2 = plsc.load_gather(rois_v, [rows, z16 + 3])
            gx1 = plsc.load_gather(gtc_v, [fgt])
            gy1 = plsc.load_gather(gtc_v, [fgt + _G])
            gx2 = plsc.load_gather(gtc_v, [fgt + 2 * _G])
            gy2 = plsc.load_gather(gtc_v, [fgt + 3 * _G])
            pw = px2 - px1 + 1.0
            ph = py2 - py1 + 1.0
            pxc = px1 + 0.5 * pw
            pyc = py1 + 0.5 * ph
            gw = gx2 - gx1 + 1.0
            gh = gy2 - gy1 + 1.0
            gxc = gx1 + 0.5 * gw
            gyc = gy1 + 0.5 * gh
            dx = (gxc - pxc) / pw
            dy = (gyc - pyc) / ph
            dw = _vlog(gw / pw)
            dh = _vlog(gh / ph)
            flat = rows * (_NUM_CLASSES * 4) + lab * 4
            plsc.store_scatter(tgt_v, [flat], dx)
            plsc.store_scatter(tgt_v, [flat + 1], dy)
            plsc.store_scatter(tgt_v, [flat + 2], dw)
            plsc.store_scatter(tgt_v, [flat + 3], dh)
            plsc.store_scatter(inw_v, [flat], one)
            plsc.store_scatter(inw_v, [flat + 1], one)
            plsc.store_scatter(inw_v, [flat + 2], one)
            plsc.store_scatter(inw_v, [flat + 3], one)

        for k in range(2, _TOTAL // 16):
            lab_v[pl.ds(k * 16, 16)] = z16
        pltpu.sync_copy(lab_v, olab_hbm)
        pltpu.sync_copy(tgt_v, otgt_hbm)
        pltpu.sync_copy(inw_v, oinw_hbm)


def kernel(rois, gt_bboxes, gt_labels):
    rois_pad = jnp.pad(rois, ((0, _NPAD - _N), (0, 0)))
    gt_cols = gt_bboxes.T.reshape(-1)
    reci, recf = _phase1(rois_pad[:, 0], rois_pad[:, 1],
                         rois_pad[:, 2], rois_pad[:, 3], gt_cols)
    frois, flab, ftgt, finw = _phase2(
        reci, recf, gt_cols, gt_labels.astype(jnp.int32))
    ftgt = ftgt.reshape(_TOTAL, _NUM_CLASSES * 4)
    finw = finw.reshape(_TOTAL, _NUM_CLASSES * 4)
    foutw = jnp.ones((_TOTAL, _NUM_CLASSES * 4), jnp.float32)
    return frois, flab, ftgt, finw, foutw
